# Initial kernel scaffold; baseline (speedup 1.0000x reference)
import jax
import jax.numpy as jnp
from jax.experimental import pallas as pl


def kernel(h, u, pos_s, pos_a, a2s_src, a2s_dst, a2s_dis, s2s_src, s2s_dst, s2s_dis, params):
    raise NotImplementedError("write your pallas kernel here")


# speedup vs baseline: 1.1450x; 1.1450x over previous
import functools

import jax
import jax.numpy as jnp
from jax.experimental import pallas as pl
from jax.experimental.pallas import tpu as pltpu

_BE = 3200
_BN = 2000
_SRCW = 144
_DSTW = 16


def _edge_mlp_body(gsrc_ref, gdst_ref, dis_ref, w0s_ref, w0d_ref, w0x_ref,
                   b0_ref, w1_ref, b1_ref, w2_ref, b2_ref, out_ref):
    gs = gsrc_ref[...].astype(jnp.bfloat16)
    gd = gdst_ref[...].astype(jnp.bfloat16)
    pre = jnp.dot(gs, w0s_ref[...], preferred_element_type=jnp.float32)
    pre = pre + jnp.dot(gd, w0d_ref[...], preferred_element_type=jnp.float32)
    pre = pre + dis_ref[...] * w0x_ref[...] + b0_ref[...]
    x = jnp.tanh(pre).astype(jnp.bfloat16)
    x = jnp.dot(x, w1_ref[...], preferred_element_type=jnp.float32) + b1_ref[...]
    x = jnp.tanh(x).astype(jnp.bfloat16)
    m = jnp.dot(x, w2_ref[...], preferred_element_type=jnp.float32) + b2_ref[...]
    be = m.shape[0]
    ones = jnp.ones((be, 1), jnp.float32)
    zeros = jnp.zeros((be, _SRCW - 129), jnp.float32)
    out_ref[...] = jnp.concatenate([m, ones, zeros], axis=1)


def _edge_mlp(gsrc, gdst, dis, w0s, w0d, w0x, b0, w1, b1, w2, b2, be=_BE):
    e = gsrc.shape[0]
    grid = (e // be,)
    wspec = lambda a: pl.BlockSpec(a.shape, lambda i: (0,) * a.ndim)
    return pl.pallas_call(
        _edge_mlp_body,
        grid=grid,
        in_specs=[
            pl.BlockSpec((be, _SRCW), lambda i: (i, 0)),
            pl.BlockSpec((be, _DSTW), lambda i: (i, 0)),
            pl.BlockSpec((be, 1), lambda i: (i, 0)),
            wspec(w0s), wspec(w0d), wspec(w0x), wspec(b0),
            wspec(w1), wspec(b1), wspec(w2), wspec(b2),
        ],
        out_specs=pl.BlockSpec((be, _SRCW), lambda i: (i, 0)),
        out_shape=jax.ShapeDtypeStruct((e, _SRCW), jnp.float32),
    )(gsrc, gdst, dis, w0s, w0d, w0x, b0, w1, b1, w2, b2)


def _node_mlp_body(ph_ref, su_ref, ss_ref, w0a_ref, w0b_ref, w0c_ref,
                   b0_ref, w1_ref, b1_ref, w2_ref, b2_ref, out_ref):
    ph = ph_ref[...].astype(jnp.bfloat16)
    su = su_ref[:, :128].astype(jnp.bfloat16)
    ss = ss_ref[...]
    cnt = jnp.maximum(ss[:, 128:129], 1.0)
    mh = (ss[:, :128] / cnt).astype(jnp.bfloat16)
    pre = jnp.dot(ph, w0a_ref[...], preferred_element_type=jnp.float32)
    pre = pre + jnp.dot(su, w0b_ref[...], preferred_element_type=jnp.float32)
    pre = pre + jnp.dot(mh, w0c_ref[...], preferred_element_type=jnp.float32)
    pre = pre + b0_ref[...]
    x = jnp.tanh(pre).astype(jnp.bfloat16)
    x = jnp.dot(x, w1_ref[...], preferred_element_type=jnp.float32) + b1_ref[...]
    x = jnp.tanh(x).astype(jnp.bfloat16)
    out_ref[...] = (jnp.dot(x, w2_ref[...], preferred_element_type=jnp.float32)
                    + b2_ref[...])


def _node_mlp(ph, su, ss, w0a, w0b, w0c, b0, w1, b1, w2, b2, bn=_BN):
    n = ph.shape[0]
    grid = (n // bn,)
    wspec = lambda a: pl.BlockSpec(a.shape, lambda i: (0,) * a.ndim)
    return pl.pallas_call(
        _node_mlp_body,
        grid=grid,
        in_specs=[
            pl.BlockSpec((bn, ph.shape[1]), lambda i: (i, 0)),
            pl.BlockSpec((bn, _SRCW), lambda i: (i, 0)),
            pl.BlockSpec((bn, _SRCW), lambda i: (i, 0)),
            wspec(w0a), wspec(w0b), wspec(w0c), wspec(b0),
            wspec(w1), wspec(b1), wspec(w2), wspec(b2),
        ],
        out_specs=pl.BlockSpec((bn, 128), lambda i: (i, 0)),
        out_shape=jax.ShapeDtypeStruct((n, 128), jnp.float32),
    )(ph, su, ss, w0a, w0b, w0c, b0, w1, b1, w2, b2)


def _edge_weights(params, name):
    w0 = params[f'{name}_W0']
    w0s = jnp.concatenate([w0[:, 5:133], w0[:, 0:2],
                           jnp.zeros((256, _SRCW - 130), jnp.float32)], axis=1).T
    w0d = jnp.concatenate([w0[:, 2:4],
                           jnp.zeros((256, _DSTW - 2), jnp.float32)], axis=1).T
    w0x = w0[:, 4:5].T
    return (w0s.astype(jnp.bfloat16), w0d.astype(jnp.bfloat16), w0x,
            params[f'{name}_b0'][None, :],
            params[f'{name}_W1'].T.astype(jnp.bfloat16),
            params[f'{name}_b1'][None, :],
            params[f'{name}_W2'].T.astype(jnp.bfloat16),
            params[f'{name}_b2'][None, :])


def kernel(h, u, pos_s, pos_a, a2s_src, a2s_dst, a2s_dis,
           s2s_src, s2s_dst, s2s_dis, params):
    n = h.shape[0]
    pad_src = jnp.zeros((n, _SRCW - 130), jnp.float32)
    pad_dst = jnp.zeros((n, _DSTW - 2), jnp.float32)
    a_tab = jnp.concatenate([u, pos_a, pad_src], axis=1)
    b_tab = jnp.concatenate([h, pos_s, pad_src], axis=1)
    p_tab = jnp.concatenate([pos_s, pad_dst], axis=1)

    m_a = _edge_mlp(a_tab[a2s_src], p_tab[a2s_dst], a2s_dis,
                    *_edge_weights(params, 'u2h'))
    sum_a = jax.ops.segment_sum(m_a, a2s_dst, num_segments=n)

    m_s = _edge_mlp(b_tab[s2s_src], p_tab[s2s_dst], s2s_dis,
                    *_edge_weights(params, 'h2h'))
    sum_s = jax.ops.segment_sum(m_s, s2s_dst, num_segments=n)

    w0 = params['upd_W0']
    ph = jnp.concatenate([pos_s, h], axis=1)
    return _node_mlp(ph, sum_a, sum_s,
                     w0[:, 0:130].T.astype(jnp.bfloat16),
                     w0[:, 130:258].T.astype(jnp.bfloat16),
                     w0[:, 258:386].T.astype(jnp.bfloat16),
                     params['upd_b0'][None, :],
                     params['upd_W1'].T.astype(jnp.bfloat16),
                     params['upd_b1'][None, :],
                     params['upd_W2'].T.astype(jnp.bfloat16),
                     params['upd_b2'][None, :])


# speedup vs baseline: 1.2740x; 1.1126x over previous
import functools

import jax
from jax import lax
import jax.numpy as jnp
from jax.experimental import pallas as pl
from jax.experimental.pallas import tpu as pltpu
from jax.experimental.pallas import tpu_sc as plsc

_BE = 3200
_BN = 2000
_SRCW = 144
_DSTW = 16
_NPAD = 10240
_SCC = 512


def _edge_mlp_body(gsrc_ref, gdst_ref, dis_ref, w0s_ref, w0d_ref, w0x_ref,
                   b0_ref, w1_ref, b1_ref, w2_ref, b2_ref, out_ref):
    gs = gsrc_ref[...].astype(jnp.bfloat16)
    gd = gdst_ref[...].astype(jnp.bfloat16)
    pre = jnp.dot(gs, w0s_ref[...], preferred_element_type=jnp.float32)
    pre = pre + jnp.dot(gd, w0d_ref[...], preferred_element_type=jnp.float32)
    pre = pre + dis_ref[...] * w0x_ref[...] + b0_ref[...]
    x = jnp.tanh(pre).astype(jnp.bfloat16)
    x = jnp.dot(x, w1_ref[...], preferred_element_type=jnp.float32) + b1_ref[...]
    x = jnp.tanh(x).astype(jnp.bfloat16)
    out_ref[...] = (jnp.dot(x, w2_ref[...], preferred_element_type=jnp.float32)
                    + b2_ref[...])


def _edge_mlp(gsrc, gdst, dis, w0s, w0d, w0x, b0, w1, b1, w2, b2, be=_BE):
    e = gsrc.shape[0]
    grid = (e // be,)
    wspec = lambda a: pl.BlockSpec(a.shape, lambda i: (0,) * a.ndim)
    return pl.pallas_call(
        _edge_mlp_body,
        grid=grid,
        in_specs=[
            pl.BlockSpec((be, _SRCW), lambda i: (i, 0)),
            pl.BlockSpec((be, _DSTW), lambda i: (i, 0)),
            pl.BlockSpec((be, 1), lambda i: (i, 0)),
            wspec(w0s), wspec(w0d), wspec(w0x), wspec(b0),
            wspec(w1), wspec(b1), wspec(w2), wspec(b2),
        ],
        out_specs=pl.BlockSpec((be, 128), lambda i: (i, 0)),
        out_shape=jax.ShapeDtypeStruct((e, 128), jnp.float32),
    )(gsrc, gdst, dis, w0s, w0d, w0x, b0, w1, b1, w2, b2)


def _node_mlp_body(ph_ref, su_ref, ss_ref, cnt_ref,
                   w0a_ref, w0b_ref, w0c_ref,
                   b0_ref, w1_ref, b1_ref, w2_ref, b2_ref, out_ref):
    ph = ph_ref[...].astype(jnp.bfloat16)
    su = su_ref[...].astype(jnp.bfloat16)
    ss = ss_ref[...]
    cnt = jnp.maximum(cnt_ref[...], 1.0)
    mh = (ss / cnt).astype(jnp.bfloat16)
    pre = jnp.dot(ph, w0a_ref[...], preferred_element_type=jnp.float32)
    pre = pre + jnp.dot(su, w0b_ref[...], preferred_element_type=jnp.float32)
    pre = pre + jnp.dot(mh, w0c_ref[...], preferred_element_type=jnp.float32)
    pre = pre + b0_ref[...]
    x = jnp.tanh(pre).astype(jnp.bfloat16)
    x = jnp.dot(x, w1_ref[...], preferred_element_type=jnp.float32) + b1_ref[...]
    x = jnp.tanh(x).astype(jnp.bfloat16)
    out_ref[...] = (jnp.dot(x, w2_ref[...], preferred_element_type=jnp.float32)
                    + b2_ref[...])


def _node_mlp(ph, su, ss, cnt, w0a, w0b, w0c, b0, w1, b1, w2, b2, bn=_BN):
    n = ph.shape[0]
    grid = (n // bn,)
    wspec = lambda a: pl.BlockSpec(a.shape, lambda i: (0,) * a.ndim)
    part = pl.BlockSpec((bn, 128), lambda i: (i, 0))
    return pl.pallas_call(
        _node_mlp_body,
        grid=grid,
        in_specs=[
            pl.BlockSpec((bn, ph.shape[1]), lambda i: (i, 0)),
            part, part,
            pl.BlockSpec((bn, 1), lambda i: (i, 0)),
            wspec(w0a), wspec(w0b), wspec(w0c), wspec(b0),
            wspec(w1), wspec(b1), wspec(w2), wspec(b2),
        ],
        out_specs=pl.BlockSpec((bn, 128), lambda i: (i, 0)),
        out_shape=jax.ShapeDtypeStruct((n, 128), jnp.float32),
    )(ph, su, ss, cnt, w0a, w0b, w0c, b0, w1, b1, w2, b2)


_HRNG = _NPAD // 2
_ACCR = _HRNG + 128


def _scatter_sum2(m_a, dst_a, m_s, dst_s, zeros_hbm):
    e = m_a.shape[0]
    n_chunks = e // _SCC
    n_steps = (n_chunks + 15) // 16
    zrows = _ACCR // 16
    orows = _HRNG // 16

    def body(ma_hbm, da_hbm, ms_hbm, ds_hbm, z_hbm, out_a, out_s,
             rows_v, idx_v, idx2_v, acc, *_):
        cid = lax.axis_index("c")
        sid = lax.axis_index("s")
        lo = cid * _HRNG
        garbage = _HRNG + sid * 8

        def zero_acc():
            pltpu.sync_copy(z_hbm.at[pl.ds(sid * zrows, zrows)],
                            acc.at[pl.ds(sid * zrows, zrows)])

        def run_set(m_hbm, dst_hbm, out):
            zero_acc()
            plsc.subcore_barrier()

            def step(k, _):
                chunk = k * 16 + sid

                @pl.when(chunk < n_chunks)
                def _():
                    base = chunk * _SCC
                    pltpu.sync_copy(dst_hbm.at[pl.ds(chunk * (_SCC // 128),
                                                     _SCC // 128)], idx_v)
                    pltpu.sync_copy(m_hbm.at[pl.ds(base, _SCC)], rows_v)
                    for j in range(_SCC // 128):
                        for l in range(8):
                            x = idx_v[j, pl.ds(l * 16, 16)]
                            y = x - lo
                            ok = (y >= 0) & (y < _HRNG)
                            idx2_v[j, pl.ds(l * 16, 16)] = jnp.where(ok, y, garbage)
                        pltpu.sync_copy(rows_v.at[pl.ds(j * 128, 128)],
                                        acc.at[idx2_v.at[j]], add=True)
                return None

            lax.fori_loop(0, n_steps, step, None)
            plsc.subcore_barrier()
            pltpu.sync_copy(acc.at[pl.ds(sid * orows, orows)],
                            out.at[pl.ds(lo + sid * orows, orows)])
            plsc.subcore_barrier()

        run_set(ma_hbm, da_hbm, out_a)
        run_set(ms_hbm, ds_hbm, out_s)

    return pl.kernel(
        body,
        out_type=(jax.ShapeDtypeStruct((_NPAD, 128), jnp.float32),
                  jax.ShapeDtypeStruct((_NPAD, 128), jnp.float32)),
        mesh=plsc.VectorSubcoreMesh(core_axis_name="c", subcore_axis_name="s"),
        scratch_types=[
            pltpu.VMEM((_SCC, 128), jnp.float32),
            pltpu.VMEM((_SCC // 128, 128), jnp.int32),
            pltpu.VMEM((_SCC // 128, 128), jnp.int32),
            pltpu.VMEM_SHARED((_ACCR, 128), jnp.float32),
        ],
    )(m_a, dst_a, m_s, dst_s, zeros_hbm)


def _edge_weights(params, name):
    w0 = params[f'{name}_W0']
    w0s = jnp.concatenate([w0[:, 5:133], w0[:, 0:2],
                           jnp.zeros((256, _SRCW - 130), jnp.float32)], axis=1).T
    w0d = jnp.concatenate([w0[:, 2:4],
                           jnp.zeros((256, _DSTW - 2), jnp.float32)], axis=1).T
    w0x = w0[:, 4:5].T
    return (w0s.astype(jnp.bfloat16), w0d.astype(jnp.bfloat16), w0x,
            params[f'{name}_b0'][None, :],
            params[f'{name}_W1'].T.astype(jnp.bfloat16),
            params[f'{name}_b1'][None, :],
            params[f'{name}_W2'].T.astype(jnp.bfloat16),
            params[f'{name}_b2'][None, :])


def kernel(h, u, pos_s, pos_a, a2s_src, a2s_dst, a2s_dis,
           s2s_src, s2s_dst, s2s_dis, params):
    n = h.shape[0]
    pad_src = jnp.zeros((n, _SRCW - 130), jnp.float32)
    pad_dst = jnp.zeros((n, _DSTW - 2), jnp.float32)
    a_tab = jnp.concatenate([u, pos_a, pad_src], axis=1)
    b_tab = jnp.concatenate([h, pos_s, pad_src], axis=1)
    p_tab = jnp.concatenate([pos_s, pad_dst], axis=1)

    e = a2s_src.shape[0]
    zeros_hbm = jnp.zeros((_ACCR, 128), jnp.float32)

    m_a = _edge_mlp(a_tab[a2s_src], p_tab[a2s_dst], a2s_dis,
                    *_edge_weights(params, 'u2h'))
    m_s = _edge_mlp(b_tab[s2s_src], p_tab[s2s_dst], s2s_dis,
                    *_edge_weights(params, 'h2h'))
    sum_a, sum_s = _scatter_sum2(m_a, a2s_dst.reshape(e // 128, 128),
                                 m_s, s2s_dst.reshape(e // 128, 128), zeros_hbm)
    cnt = jax.ops.segment_sum(jnp.ones((e,), jnp.float32), s2s_dst,
                              num_segments=n)[:, None]

    w0 = params['upd_W0']
    ph = jnp.concatenate([pos_s, h], axis=1)
    return _node_mlp(ph, sum_a, sum_s, cnt,
                     w0[:, 0:130].T.astype(jnp.bfloat16),
                     w0[:, 130:258].T.astype(jnp.bfloat16),
                     w0[:, 258:386].T.astype(jnp.bfloat16),
                     params['upd_b0'][None, :],
                     params['upd_W1'].T.astype(jnp.bfloat16),
                     params['upd_b1'][None, :],
                     params['upd_W2'].T.astype(jnp.bfloat16),
                     params['upd_b2'][None, :])


# speedup vs baseline: 2.9903x; 2.3472x over previous
import functools

import jax
from jax import lax
import jax.numpy as jnp
from jax.experimental import pallas as pl
from jax.experimental.pallas import tpu as pltpu
from jax.experimental.pallas import tpu_sc as plsc

_BE = 3200
_BN = 2000
_SRCW = 144
_DSTW = 16
_NPAD = 10240
_SCC = 512


def _edge_mlp_body(gsrc_ref, gdst_ref, dis_ref, w0s_ref, w0d_ref, w0x_ref,
                   b0_ref, w1_ref, b1_ref, w2_ref, b2_ref, out_ref):
    gs = gsrc_ref[...].astype(jnp.bfloat16)
    gd = gdst_ref[...].astype(jnp.bfloat16)
    pre = jnp.dot(gs, w0s_ref[...], preferred_element_type=jnp.float32)
    pre = pre + jnp.dot(gd, w0d_ref[...], preferred_element_type=jnp.float32)
    pre = pre + dis_ref[...] * w0x_ref[...] + b0_ref[...]
    x = jnp.tanh(pre).astype(jnp.bfloat16)
    x = jnp.dot(x, w1_ref[...], preferred_element_type=jnp.float32) + b1_ref[...]
    x = jnp.tanh(x).astype(jnp.bfloat16)
    out_ref[...] = (jnp.dot(x, w2_ref[...], preferred_element_type=jnp.float32)
                    + b2_ref[...])


def _edge_mlp(gsrc, gdst, dis, w0s, w0d, w0x, b0, w1, b1, w2, b2, be=_BE):
    e = gsrc.shape[0]
    grid = (e // be,)
    wspec = lambda a: pl.BlockSpec(a.shape, lambda i: (0,) * a.ndim)
    return pl.pallas_call(
        _edge_mlp_body,
        grid=grid,
        in_specs=[
            pl.BlockSpec((be, 128), lambda i: (i, 0)),
            pl.BlockSpec((be, 8), lambda i: (i, 0)),
            pl.BlockSpec((be, 1), lambda i: (i, 0)),
            wspec(w0s), wspec(w0d), wspec(w0x), wspec(b0),
            wspec(w1), wspec(b1), wspec(w2), wspec(b2),
        ],
        out_specs=pl.BlockSpec((be, 128), lambda i: (i, 0)),
        out_shape=jax.ShapeDtypeStruct((e, 128), jnp.float32),
    )(gsrc, gdst, dis, w0s, w0d, w0x, b0, w1, b1, w2, b2)


def _node_mlp_body(ph_ref, su_ref, ss_ref, cnt_ref,
                   w0a_ref, w0b_ref, w0c_ref,
                   b0_ref, w1_ref, b1_ref, w2_ref, b2_ref, out_ref):
    ph = ph_ref[...].astype(jnp.bfloat16)
    su = su_ref[...].astype(jnp.bfloat16)
    ss = ss_ref[...]
    cnt = jnp.maximum(cnt_ref[...], 1.0)
    mh = (ss / cnt).astype(jnp.bfloat16)
    pre = jnp.dot(ph, w0a_ref[...], preferred_element_type=jnp.float32)
    pre = pre + jnp.dot(su, w0b_ref[...], preferred_element_type=jnp.float32)
    pre = pre + jnp.dot(mh, w0c_ref[...], preferred_element_type=jnp.float32)
    pre = pre + b0_ref[...]
    x = jnp.tanh(pre).astype(jnp.bfloat16)
    x = jnp.dot(x, w1_ref[...], preferred_element_type=jnp.float32) + b1_ref[...]
    x = jnp.tanh(x).astype(jnp.bfloat16)
    out_ref[...] = (jnp.dot(x, w2_ref[...], preferred_element_type=jnp.float32)
                    + b2_ref[...])


def _node_mlp(ph, su, ss, cnt, w0a, w0b, w0c, b0, w1, b1, w2, b2, bn=_BN):
    n = ph.shape[0]
    grid = (n // bn,)
    wspec = lambda a: pl.BlockSpec(a.shape, lambda i: (0,) * a.ndim)
    part = pl.BlockSpec((bn, 128), lambda i: (i, 0))
    return pl.pallas_call(
        _node_mlp_body,
        grid=grid,
        in_specs=[
            pl.BlockSpec((bn, ph.shape[1]), lambda i: (i, 0)),
            part, part,
            pl.BlockSpec((bn, 1), lambda i: (i, 0)),
            wspec(w0a), wspec(w0b), wspec(w0c), wspec(b0),
            wspec(w1), wspec(b1), wspec(w2), wspec(b2),
        ],
        out_specs=pl.BlockSpec((bn, 128), lambda i: (i, 0)),
        out_shape=jax.ShapeDtypeStruct((n, 128), jnp.float32),
    )(ph, su, ss, cnt, w0a, w0b, w0c, b0, w1, b1, w2, b2)


_P2N = 40960


def _gather_edges(feat_tab, pos2, src2d, dst2d):
    e = src2d.shape[0]
    n_chunks = e // _SCC
    n_steps = (n_chunks + 31) // 32

    def body(tab_hbm, pos2_hbm, src_hbm, dst_hbm, gsrc_out, gpos_out,
             rows_v, pbuf_v, idxs_v, idxd_v, ptab_v, sem, *_):
        cid = lax.axis_index("c")
        sid = lax.axis_index("s")
        wid = sid * 2 + cid
        pltpu.sync_copy(pos2_hbm, ptab_v)

        def zstep(i, _):
            pbuf_v[pl.ds(i * 16, 16)] = jnp.zeros((16,), jnp.float32)
            return None

        lax.fori_loop(0, _SCC * 8 // 16, zstep, None)
        lane8 = jax.lax.iota(jnp.int32, 16) * 8

        def step(k, _):
            chunk = k * 32 + wid

            @pl.when(chunk < n_chunks)
            def _():
                base = chunk * _SCC
                pltpu.sync_copy(src_hbm.at[pl.ds(base, _SCC)], idxs_v)
                pltpu.sync_copy(dst_hbm.at[pl.ds(base, _SCC)], idxd_v)
                copies = [pltpu.make_async_copy(
                    tab_hbm.at[idxs_v.at[pl.ds(j * 128, 128)]],
                    rows_v.at[pl.ds(j * 128, 128)], sem)
                    for j in range(_SCC // 128)]
                for c in copies:
                    c.start()
                for g in range(_SCC // 16):
                    si = idxs_v[pl.ds(g * 16, 16)] * 2
                    di = idxd_v[pl.ds(g * 16, 16)] * 2 + _P2N // 2
                    off = g * 128 + lane8
                    plsc.store_scatter(pbuf_v, [off],
                                       plsc.load_gather(ptab_v, [si]))
                    plsc.store_scatter(pbuf_v, [off + 1],
                                       plsc.load_gather(ptab_v, [si + 1]))
                    plsc.store_scatter(pbuf_v, [off + 2],
                                       plsc.load_gather(ptab_v, [di]))
                    plsc.store_scatter(pbuf_v, [off + 3],
                                       plsc.load_gather(ptab_v, [di + 1]))
                pltpu.sync_copy(pbuf_v, gpos_out.at[pl.ds(base * 8, _SCC * 8)])
                for c in copies:
                    c.wait()
                pltpu.sync_copy(rows_v, gsrc_out.at[pl.ds(base, _SCC)])
            return None

        lax.fori_loop(0, n_steps, step, None)

    return pl.kernel(
        body,
        out_type=(jax.ShapeDtypeStruct((e, 128), jnp.float32),
                  jax.ShapeDtypeStruct((e * 8,), jnp.float32)),
        mesh=plsc.VectorSubcoreMesh(core_axis_name="c", subcore_axis_name="s"),
        compiler_params=pltpu.CompilerParams(needs_layout_passes=False),
        scratch_types=[
            pltpu.VMEM((_SCC, 128), jnp.float32),
            pltpu.VMEM((_SCC * 8,), jnp.float32),
            pltpu.VMEM((_SCC,), jnp.int32),
            pltpu.VMEM((_SCC,), jnp.int32),
            pltpu.VMEM((_P2N,), jnp.float32),
            pltpu.SemaphoreType.DMA,
        ],
    )(feat_tab, pos2, src2d, dst2d)


_HRNG = _NPAD // 2
_ACCR = _HRNG + 128


def _scatter_sum2(m_a, dst_a, m_s, dst_s, zeros_hbm):
    e = m_a.shape[0]
    n_chunks = e // _SCC
    n_steps = (n_chunks + 15) // 16
    zrows = _ACCR // 16
    orows = _HRNG // 16

    def body(ma_hbm, da_hbm, ms_hbm, ds_hbm, z_hbm, out_a, out_s,
             rows_v, idx_v, idx2_v, acc, *_):
        cid = lax.axis_index("c")
        sid = lax.axis_index("s")
        lo = cid * _HRNG
        garbage = _HRNG + sid * 8

        def zero_acc():
            pltpu.sync_copy(z_hbm.at[pl.ds(sid * zrows, zrows)],
                            acc.at[pl.ds(sid * zrows, zrows)])

        def run_set(m_hbm, dst_hbm, out):
            zero_acc()
            plsc.subcore_barrier()

            def step(k, _):
                chunk = k * 16 + sid

                @pl.when(chunk < n_chunks)
                def _():
                    base = chunk * _SCC
                    pltpu.sync_copy(dst_hbm.at[pl.ds(chunk * (_SCC // 128),
                                                     _SCC // 128)], idx_v)
                    pltpu.sync_copy(m_hbm.at[pl.ds(base, _SCC)], rows_v)
                    for j in range(_SCC // 128):
                        for l in range(8):
                            x = idx_v[j, pl.ds(l * 16, 16)]
                            y = x - lo
                            ok = (y >= 0) & (y < _HRNG)
                            idx2_v[j, pl.ds(l * 16, 16)] = jnp.where(ok, y, garbage)
                        pltpu.sync_copy(rows_v.at[pl.ds(j * 128, 128)],
                                        acc.at[idx2_v.at[j]], add=True)
                return None

            lax.fori_loop(0, n_steps, step, None)
            plsc.subcore_barrier()
            pltpu.sync_copy(acc.at[pl.ds(sid * orows, orows)],
                            out.at[pl.ds(lo + sid * orows, orows)])
            plsc.subcore_barrier()

        run_set(ma_hbm, da_hbm, out_a)
        run_set(ms_hbm, ds_hbm, out_s)

    return pl.kernel(
        body,
        out_type=(jax.ShapeDtypeStruct((_NPAD, 128), jnp.float32),
                  jax.ShapeDtypeStruct((_NPAD, 128), jnp.float32)),
        mesh=plsc.VectorSubcoreMesh(core_axis_name="c", subcore_axis_name="s"),
        scratch_types=[
            pltpu.VMEM((_SCC, 128), jnp.float32),
            pltpu.VMEM((_SCC // 128, 128), jnp.int32),
            pltpu.VMEM((_SCC // 128, 128), jnp.int32),
            pltpu.VMEM_SHARED((_ACCR, 128), jnp.float32),
        ],
    )(m_a, dst_a, m_s, dst_s, zeros_hbm)


def _edge_weights(params, name):
    w0 = params[f'{name}_W0']
    w0s = w0[:, 5:133].T
    w0d = jnp.concatenate([w0[:, 0:4],
                           jnp.zeros((256, 4), jnp.float32)], axis=1).T
    w0x = w0[:, 4:5].T
    return (w0s.astype(jnp.bfloat16), w0d.astype(jnp.bfloat16), w0x,
            params[f'{name}_b0'][None, :],
            params[f'{name}_W1'].T.astype(jnp.bfloat16),
            params[f'{name}_b1'][None, :],
            params[f'{name}_W2'].T.astype(jnp.bfloat16),
            params[f'{name}_b2'][None, :])


def kernel(h, u, pos_s, pos_a, a2s_src, a2s_dst, a2s_dis,
           s2s_src, s2s_dst, s2s_dis, params):
    n = h.shape[0]
    e = a2s_src.shape[0]
    zeros_hbm = jnp.zeros((_ACCR, 128), jnp.float32)
    pad2n = jnp.zeros((_P2N // 2 - 2 * n,), jnp.float32)
    pos_sf = pos_s.reshape(-1)
    pos2_a = jnp.concatenate([pos_a.reshape(-1), pad2n, pos_sf, pad2n])
    pos2_s = jnp.concatenate([pos_sf, pad2n, pos_sf, pad2n])

    ga_src, ga_pos = _gather_edges(u, pos2_a, a2s_src, a2s_dst)
    m_a = _edge_mlp(ga_src, ga_pos.reshape(e, 8), a2s_dis,
                    *_edge_weights(params, 'u2h'))
    gs_src, gs_pos = _gather_edges(h, pos2_s, s2s_src, s2s_dst)
    m_s = _edge_mlp(gs_src, gs_pos.reshape(e, 8), s2s_dis,
                    *_edge_weights(params, 'h2h'))
    sum_a, sum_s = _scatter_sum2(m_a, a2s_dst.reshape(e // 128, 128),
                                 m_s, s2s_dst.reshape(e // 128, 128), zeros_hbm)
    cnt = jax.ops.segment_sum(jnp.ones((e,), jnp.float32), s2s_dst,
                              num_segments=n)[:, None]

    w0 = params['upd_W0']
    ph = jnp.concatenate([pos_s, h], axis=1)
    return _node_mlp(ph, sum_a, sum_s, cnt,
                     w0[:, 0:130].T.astype(jnp.bfloat16),
                     w0[:, 130:258].T.astype(jnp.bfloat16),
                     w0[:, 258:386].T.astype(jnp.bfloat16),
                     params['upd_b0'][None, :],
                     params['upd_W1'].T.astype(jnp.bfloat16),
                     params['upd_b1'][None, :],
                     params['upd_W2'].T.astype(jnp.bfloat16),
                     params['upd_b2'][None, :])


# speedup vs baseline: 3.4265x; 1.1459x over previous
import functools

import jax
from jax import lax
import jax.numpy as jnp
from jax.experimental import pallas as pl
from jax.experimental.pallas import tpu as pltpu
from jax.experimental.pallas import tpu_sc as plsc

_BE = 3200
_BN = 2000
_SRCW = 144
_DSTW = 16
_NPAD = 10240
_SCC = 512


def _edge_mlp_body(gsrc_ref, gdst_ref, dis_ref, w0s_ref, w0d_ref, w0x_ref,
                   b0_ref, w1_ref, b1_ref, w2_ref, b2_ref, out_ref):
    gs = gsrc_ref[...].astype(jnp.bfloat16)
    gd = gdst_ref[...].astype(jnp.bfloat16)
    pre = jnp.dot(gs, w0s_ref[...], preferred_element_type=jnp.float32)
    pre = pre + jnp.dot(gd, w0d_ref[...], preferred_element_type=jnp.float32)
    pre = pre + dis_ref[...] * w0x_ref[...] + b0_ref[...]
    x = jnp.tanh(pre).astype(jnp.bfloat16)
    x = jnp.dot(x, w1_ref[...], preferred_element_type=jnp.float32) + b1_ref[...]
    x = jnp.tanh(x).astype(jnp.bfloat16)
    out_ref[...] = (jnp.dot(x, w2_ref[...], preferred_element_type=jnp.float32)
                    + b2_ref[...])


def _edge_mlp(gsrc, gdst, dis, w0s, w0d, w0x, b0, w1, b1, w2, b2, be=_BE):
    e = gsrc.shape[0]
    grid = (e // be,)
    wspec = lambda a: pl.BlockSpec(a.shape, lambda i: (0,) * a.ndim)
    return pl.pallas_call(
        _edge_mlp_body,
        grid=grid,
        in_specs=[
            pl.BlockSpec((be, 128), lambda i: (i, 0)),
            pl.BlockSpec((be, 8), lambda i: (i, 0)),
            pl.BlockSpec((be, 1), lambda i: (i, 0)),
            wspec(w0s), wspec(w0d), wspec(w0x), wspec(b0),
            wspec(w1), wspec(b1), wspec(w2), wspec(b2),
        ],
        out_specs=pl.BlockSpec((be, 128), lambda i: (i, 0)),
        out_shape=jax.ShapeDtypeStruct((e, 128), jnp.float32),
    )(gsrc, gdst, dis, w0s, w0d, w0x, b0, w1, b1, w2, b2)


def _node_mlp_body(ph_ref, su_ref, ss_ref, cnt_ref,
                   w0a_ref, w0b_ref, w0c_ref,
                   b0_ref, w1_ref, b1_ref, w2_ref, b2_ref, out_ref):
    ph = ph_ref[...].astype(jnp.bfloat16)
    su = su_ref[...].astype(jnp.bfloat16)
    ss = ss_ref[...]
    cnt = jnp.maximum(cnt_ref[...], 1.0)
    mh = (ss / cnt).astype(jnp.bfloat16)
    pre = jnp.dot(ph, w0a_ref[...], preferred_element_type=jnp.float32)
    pre = pre + jnp.dot(su, w0b_ref[...], preferred_element_type=jnp.float32)
    pre = pre + jnp.dot(mh, w0c_ref[...], preferred_element_type=jnp.float32)
    pre = pre + b0_ref[...]
    x = jnp.tanh(pre).astype(jnp.bfloat16)
    x = jnp.dot(x, w1_ref[...], preferred_element_type=jnp.float32) + b1_ref[...]
    x = jnp.tanh(x).astype(jnp.bfloat16)
    out_ref[...] = (jnp.dot(x, w2_ref[...], preferred_element_type=jnp.float32)
                    + b2_ref[...])


def _node_mlp(ph, su, ss, cnt, w0a, w0b, w0c, b0, w1, b1, w2, b2, bn=_BN):
    n = ph.shape[0]
    grid = (n // bn,)
    wspec = lambda a: pl.BlockSpec(a.shape, lambda i: (0,) * a.ndim)
    part = pl.BlockSpec((bn, 128), lambda i: (i, 0))
    return pl.pallas_call(
        _node_mlp_body,
        grid=grid,
        in_specs=[
            pl.BlockSpec((bn, ph.shape[1]), lambda i: (i, 0)),
            part, part,
            pl.BlockSpec((bn, 1), lambda i: (i, 0)),
            wspec(w0a), wspec(w0b), wspec(w0c), wspec(b0),
            wspec(w1), wspec(b1), wspec(w2), wspec(b2),
        ],
        out_specs=pl.BlockSpec((bn, 128), lambda i: (i, 0)),
        out_shape=jax.ShapeDtypeStruct((n, 128), jnp.float32),
    )(ph, su, ss, cnt, w0a, w0b, w0c, b0, w1, b1, w2, b2)


_P2N = 40960


def _gather_edges(feat_tab, pos2, src1d, dst1d, count_dst=False):
    e = src1d.shape[0]
    n_chunks = e // _SCC
    n_steps = (n_chunks + 31) // 32
    crows = _HRNG // 16

    def body(tab_hbm, pos2_hbm, src_hbm, dst_hbm, gsrc_out, gpos_out,
             *rest):
        if count_dst:
            (cnt_out, rows_v, pbuf_v, idxs_v, idxd_v, ptab_v,
             cnt_v, tmp_v, facc_v, stage, sem) = rest
        else:
            rows_v, pbuf_v, idxs_v, idxd_v, ptab_v, sem = rest
        cid = lax.axis_index("c")
        sid = lax.axis_index("s")
        wid = sid * 2 + cid
        pltpu.sync_copy(pos2_hbm, ptab_v)

        def zstep(i, _):
            pbuf_v[pl.ds(i * 16, 16)] = jnp.zeros((16,), jnp.float32)
            return None

        lax.fori_loop(0, _SCC * 8 // 16, zstep, None)
        if count_dst:
            def czstep(i, _):
                cnt_v[pl.ds(i * 16, 16)] = jnp.zeros((16,), jnp.float32)
                return None

            lax.fori_loop(0, (_HRNG + 16) // 16, czstep, None)
        lane8 = jax.lax.iota(jnp.int32, 16) * 8

        def step(k, _):
            chunk = k * 32 + wid

            @pl.when(chunk < n_chunks)
            def _():
                base = chunk * _SCC
                pltpu.sync_copy(src_hbm.at[pl.ds(base, _SCC)], idxs_v)
                pltpu.sync_copy(dst_hbm.at[pl.ds(base, _SCC)], idxd_v)
                copies = [pltpu.make_async_copy(
                    tab_hbm.at[idxs_v.at[pl.ds(j * 128, 128)]],
                    rows_v.at[pl.ds(j * 128, 128)], sem)
                    for j in range(_SCC // 128)]
                for c in copies:
                    c.start()
                for g in range(_SCC // 16):
                    si = idxs_v[pl.ds(g * 16, 16)] * 2
                    di = idxd_v[pl.ds(g * 16, 16)] * 2 + _P2N // 2
                    off = g * 128 + lane8
                    plsc.store_scatter(pbuf_v, [off],
                                       plsc.load_gather(ptab_v, [si]))
                    plsc.store_scatter(pbuf_v, [off + 1],
                                       plsc.load_gather(ptab_v, [si + 1]))
                    plsc.store_scatter(pbuf_v, [off + 2],
                                       plsc.load_gather(ptab_v, [di]))
                    plsc.store_scatter(pbuf_v, [off + 3],
                                       plsc.load_gather(ptab_v, [di + 1]))

                pltpu.sync_copy(pbuf_v, gpos_out.at[pl.ds(base * 8, _SCC * 8)])
                for c in copies:
                    c.wait()
                pltpu.sync_copy(rows_v, gsrc_out.at[pl.ds(base, _SCC)])
            return None

        lax.fori_loop(0, n_steps, step, None)

        if count_dst:
            def cstep(k, _):
                chunk = k * 16 + sid

                @pl.when(chunk < n_chunks)
                def _():
                    pltpu.sync_copy(dst_hbm.at[pl.ds(chunk * _SCC, _SCC)],
                                    idxd_v)
                    for g in range(_SCC // 16):
                        dv = idxd_v[pl.ds(g * 16, 16)]
                        yc = dv - cid * _HRNG
                        yc = jnp.where((yc >= 0) & (yc < _HRNG), yc,
                                       _HRNG + sid)
                        crun, clast = plsc.scan_count(yc)
                        plsc.addupdate_scatter(cnt_v, [yc],
                                               crun.astype(jnp.float32),
                                               mask=clast)
                return None

            lax.fori_loop(0, (n_chunks + 15) // 16, cstep, None)
            pltpu.sync_copy(cnt_v.at[pl.ds(0, _HRNG)],
                            stage.at[pl.ds(sid * _HRNG, _HRNG)])
            plsc.subcore_barrier()

            def fzstep(i, _):
                facc_v[pl.ds(i * 16, 16)] = jnp.zeros((16,), jnp.float32)
                return None

            lax.fori_loop(0, crows // 16, fzstep, None)
            for t in range(16):
                pltpu.sync_copy(
                    stage.at[pl.ds(t * _HRNG + sid * crows, crows)], tmp_v)

                def astep(i, _):
                    facc_v[pl.ds(i * 16, 16)] = (facc_v[pl.ds(i * 16, 16)]
                                                 + tmp_v[pl.ds(i * 16, 16)])
                    return None

                lax.fori_loop(0, crows // 16, astep, None)
            pltpu.sync_copy(
                facc_v, cnt_out.at[pl.ds(cid * _HRNG + sid * crows, crows)])

    outs = [jax.ShapeDtypeStruct((e, 128), jnp.float32),
            jax.ShapeDtypeStruct((e * 8,), jnp.float32)]
    scratch = [
        pltpu.VMEM((_SCC, 128), jnp.float32),
        pltpu.VMEM((_SCC * 8,), jnp.float32),
        pltpu.VMEM((_SCC,), jnp.int32),
        pltpu.VMEM((_SCC,), jnp.int32),
        pltpu.VMEM((_P2N,), jnp.float32),
    ]
    if count_dst:
        outs.append(jax.ShapeDtypeStruct((_NPAD,), jnp.float32))
        scratch += [
            pltpu.VMEM((_HRNG + 16,), jnp.float32),
            pltpu.VMEM((crows,), jnp.float32),
            pltpu.VMEM((crows,), jnp.float32),
            pltpu.VMEM_SHARED((16 * _HRNG,), jnp.float32),
        ]
    scratch.append(pltpu.SemaphoreType.DMA)
    return pl.kernel(
        body,
        out_type=tuple(outs),
        mesh=plsc.VectorSubcoreMesh(core_axis_name="c", subcore_axis_name="s"),
        compiler_params=pltpu.CompilerParams(needs_layout_passes=False),
        scratch_types=scratch,
    )(feat_tab, pos2, src1d, dst1d)


_HRNG = _NPAD // 2
_ACCR = _HRNG + 128


def _scatter_sum2(m_a, dst_a, m_s, dst_s, zeros_hbm):
    e = m_a.shape[0]
    n_chunks = e // _SCC
    n_steps = (n_chunks + 15) // 16
    zrows = _ACCR // 16
    orows = _HRNG // 16

    def body(ma_hbm, da_hbm, ms_hbm, ds_hbm, z_hbm, out_a, out_s,
             rows_v, idx_v, idx2_v, acc, *_):
        cid = lax.axis_index("c")
        sid = lax.axis_index("s")
        lo = cid * _HRNG
        garbage = _HRNG + sid * 8

        def zero_acc():
            pltpu.sync_copy(z_hbm.at[pl.ds(sid * zrows, zrows)],
                            acc.at[pl.ds(sid * zrows, zrows)])

        def run_set(m_hbm, dst_hbm, out):
            zero_acc()
            plsc.subcore_barrier()

            def step(k, _):
                chunk = k * 16 + sid

                @pl.when(chunk < n_chunks)
                def _():
                    base = chunk * _SCC
                    pltpu.sync_copy(dst_hbm.at[pl.ds(chunk * (_SCC // 128),
                                                     _SCC // 128)], idx_v)
                    pltpu.sync_copy(m_hbm.at[pl.ds(base, _SCC)], rows_v)
                    for j in range(_SCC // 128):
                        for l in range(8):
                            x = idx_v[j, pl.ds(l * 16, 16)]
                            y = x - lo
                            ok = (y >= 0) & (y < _HRNG)
                            idx2_v[j, pl.ds(l * 16, 16)] = jnp.where(ok, y, garbage)
                        pltpu.sync_copy(rows_v.at[pl.ds(j * 128, 128)],
                                        acc.at[idx2_v.at[j]], add=True)
                return None

            lax.fori_loop(0, n_steps, step, None)
            plsc.subcore_barrier()
            pltpu.sync_copy(acc.at[pl.ds(sid * orows, orows)],
                            out.at[pl.ds(lo + sid * orows, orows)])
            plsc.subcore_barrier()

        run_set(ma_hbm, da_hbm, out_a)
        run_set(ms_hbm, ds_hbm, out_s)

    return pl.kernel(
        body,
        out_type=(jax.ShapeDtypeStruct((_NPAD, 128), jnp.float32),
                  jax.ShapeDtypeStruct((_NPAD, 128), jnp.float32)),
        mesh=plsc.VectorSubcoreMesh(core_axis_name="c", subcore_axis_name="s"),
        scratch_types=[
            pltpu.VMEM((_SCC, 128), jnp.float32),
            pltpu.VMEM((_SCC // 128, 128), jnp.int32),
            pltpu.VMEM((_SCC // 128, 128), jnp.int32),
            pltpu.VMEM_SHARED((_ACCR, 128), jnp.float32),
        ],
    )(m_a, dst_a, m_s, dst_s, zeros_hbm)


def _edge_weights(params, name):
    w0 = params[f'{name}_W0']
    w0s = w0[:, 5:133].T
    w0d = jnp.concatenate([w0[:, 0:4],
                           jnp.zeros((256, 4), jnp.float32)], axis=1).T
    w0x = w0[:, 4:5].T
    return (w0s.astype(jnp.bfloat16), w0d.astype(jnp.bfloat16), w0x,
            params[f'{name}_b0'][None, :],
            params[f'{name}_W1'].T.astype(jnp.bfloat16),
            params[f'{name}_b1'][None, :],
            params[f'{name}_W2'].T.astype(jnp.bfloat16),
            params[f'{name}_b2'][None, :])


def kernel(h, u, pos_s, pos_a, a2s_src, a2s_dst, a2s_dis,
           s2s_src, s2s_dst, s2s_dis, params):
    n = h.shape[0]
    e = a2s_src.shape[0]
    zeros_hbm = jnp.zeros((_ACCR, 128), jnp.float32)
    pad2n = jnp.zeros((_P2N // 2 - 2 * n,), jnp.float32)
    pos_sf = pos_s.reshape(-1)
    pos2_a = jnp.concatenate([pos_a.reshape(-1), pad2n, pos_sf, pad2n])
    pos2_s = jnp.concatenate([pos_sf, pad2n, pos_sf, pad2n])

    ga_src, ga_pos = _gather_edges(u, pos2_a, a2s_src, a2s_dst)
    m_a = _edge_mlp(ga_src, ga_pos.reshape(e, 8), a2s_dis,
                    *_edge_weights(params, 'u2h'))
    gs_src, gs_pos, cnt2 = _gather_edges(h, pos2_s, s2s_src, s2s_dst,
                                         count_dst=True)
    m_s = _edge_mlp(gs_src, gs_pos.reshape(e, 8), s2s_dis,
                    *_edge_weights(params, 'h2h'))
    sum_a, sum_s = _scatter_sum2(m_a, a2s_dst.reshape(e // 128, 128),
                                 m_s, s2s_dst.reshape(e // 128, 128), zeros_hbm)
    cnt = cnt2[:n, None]

    w0 = params['upd_W0']
    ph = jnp.concatenate([pos_s, h], axis=1)
    return _node_mlp(ph, sum_a, sum_s, cnt,
                     w0[:, 0:130].T.astype(jnp.bfloat16),
                     w0[:, 130:258].T.astype(jnp.bfloat16),
                     w0[:, 258:386].T.astype(jnp.bfloat16),
                     params['upd_b0'][None, :],
                     params['upd_W1'].T.astype(jnp.bfloat16),
                     params['upd_b1'][None, :],
                     params['upd_W2'].T.astype(jnp.bfloat16),
                     params['upd_b2'][None, :])


# speedup vs baseline: 3.6295x; 1.0592x over previous
import functools

import jax
from jax import lax
import jax.numpy as jnp
from jax.experimental import pallas as pl
from jax.experimental.pallas import tpu as pltpu
from jax.experimental.pallas import tpu_sc as plsc

_BE = 3200
_BN = 2000
_SRCW = 144
_DSTW = 16
_NPAD = 10240
_SCC = 512
_SCB = 256


def _edge_mlp_body(gsrc_ref, gdst_ref, dis_ref, w0s_ref, w0d_ref, w0x_ref,
                   b0_ref, w1_ref, b1_ref, w2_ref, b2_ref, out_ref):
    gs = gsrc_ref[...].astype(jnp.bfloat16)
    gd = gdst_ref[...].astype(jnp.bfloat16)
    pre = jnp.dot(gs, w0s_ref[...], preferred_element_type=jnp.float32)
    pre = pre + jnp.dot(gd, w0d_ref[...], preferred_element_type=jnp.float32)
    pre = pre + dis_ref[...] * w0x_ref[...] + b0_ref[...]
    x = jnp.tanh(pre).astype(jnp.bfloat16)
    x = jnp.dot(x, w1_ref[...], preferred_element_type=jnp.float32) + b1_ref[...]
    x = jnp.tanh(x).astype(jnp.bfloat16)
    out_ref[...] = (jnp.dot(x, w2_ref[...], preferred_element_type=jnp.float32)
                    + b2_ref[...])


def _edge_mlp(gsrc, gdst, dis, w0s, w0d, w0x, b0, w1, b1, w2, b2, be=_BE):
    e = gsrc.shape[0]
    grid = (e // be,)
    wspec = lambda a: pl.BlockSpec(a.shape, lambda i: (0,) * a.ndim)
    return pl.pallas_call(
        _edge_mlp_body,
        grid=grid,
        in_specs=[
            pl.BlockSpec((be, 128), lambda i: (i, 0)),
            pl.BlockSpec((be, 8), lambda i: (i, 0)),
            pl.BlockSpec((be, 1), lambda i: (i, 0)),
            wspec(w0s), wspec(w0d), wspec(w0x), wspec(b0),
            wspec(w1), wspec(b1), wspec(w2), wspec(b2),
        ],
        out_specs=pl.BlockSpec((be, 128), lambda i: (i, 0)),
        out_shape=jax.ShapeDtypeStruct((e, 128), jnp.float32),
    )(gsrc, gdst, dis, w0s, w0d, w0x, b0, w1, b1, w2, b2)


def _node_mlp_body(ph_ref, su_ref, ss_ref, cnt_ref,
                   w0a_ref, w0b_ref, w0c_ref,
                   b0_ref, w1_ref, b1_ref, w2_ref, b2_ref, out_ref):
    ph = ph_ref[...].astype(jnp.bfloat16)
    su = su_ref[...].astype(jnp.bfloat16)
    ss = ss_ref[...]
    cnt = jnp.maximum(cnt_ref[...], 1.0)
    mh = (ss / cnt).astype(jnp.bfloat16)
    pre = jnp.dot(ph, w0a_ref[...], preferred_element_type=jnp.float32)
    pre = pre + jnp.dot(su, w0b_ref[...], preferred_element_type=jnp.float32)
    pre = pre + jnp.dot(mh, w0c_ref[...], preferred_element_type=jnp.float32)
    pre = pre + b0_ref[...]
    x = jnp.tanh(pre).astype(jnp.bfloat16)
    x = jnp.dot(x, w1_ref[...], preferred_element_type=jnp.float32) + b1_ref[...]
    x = jnp.tanh(x).astype(jnp.bfloat16)
    out_ref[...] = (jnp.dot(x, w2_ref[...], preferred_element_type=jnp.float32)
                    + b2_ref[...])


def _node_mlp(ph, su, ss, cnt, w0a, w0b, w0c, b0, w1, b1, w2, b2, bn=_BN):
    n = ph.shape[0]
    grid = (n // bn,)
    wspec = lambda a: pl.BlockSpec(a.shape, lambda i: (0,) * a.ndim)
    part = pl.BlockSpec((bn, 128), lambda i: (i, 0))
    return pl.pallas_call(
        _node_mlp_body,
        grid=grid,
        in_specs=[
            pl.BlockSpec((bn, ph.shape[1]), lambda i: (i, 0)),
            part, part,
            pl.BlockSpec((bn, 1), lambda i: (i, 0)),
            wspec(w0a), wspec(w0b), wspec(w0c), wspec(b0),
            wspec(w1), wspec(b1), wspec(w2), wspec(b2),
        ],
        out_specs=pl.BlockSpec((bn, 128), lambda i: (i, 0)),
        out_shape=jax.ShapeDtypeStruct((n, 128), jnp.float32),
    )(ph, su, ss, cnt, w0a, w0b, w0c, b0, w1, b1, w2, b2)


_P2N = 40960


def _gather_edges(feat_tab, pos2, src1d, dst1d, count_dst=False):
    e = src1d.shape[0]
    n_chunks = e // _SCC
    n_steps = (n_chunks + 31) // 32
    crows = _HRNG // 16

    def body(tab_hbm, pos2_hbm, src_hbm, dst_hbm, gsrc_out, gpos_out,
             *rest):
        if count_dst:
            (cnt_out, rows_v, pbuf_v, idxs_v, idxd_v, ptab_v,
             cnt_v, tmp_v, facc_v, stage, sem) = rest
        else:
            rows_v, pbuf_v, idxs_v, idxd_v, ptab_v, sem = rest
        cid = lax.axis_index("c")
        sid = lax.axis_index("s")
        wid = sid * 2 + cid
        pltpu.sync_copy(pos2_hbm, ptab_v)

        def zstep(i, _):
            pbuf_v[pl.ds(i * 16, 16)] = jnp.zeros((16,), jnp.float32)
            return None

        lax.fori_loop(0, _SCC * 8 // 16, zstep, None)
        if count_dst:
            def czstep(i, _):
                cnt_v[pl.ds(i * 16, 16)] = jnp.zeros((16,), jnp.float32)
                return None

            lax.fori_loop(0, (_HRNG + 16) // 16, czstep, None)
        lane8 = jax.lax.iota(jnp.int32, 16) * 8

        def step(k, _):
            chunk = k * 32 + wid

            @pl.when(chunk < n_chunks)
            def _():
                base = chunk * _SCC
                pltpu.sync_copy(src_hbm.at[pl.ds(base, _SCC)], idxs_v)
                pltpu.sync_copy(dst_hbm.at[pl.ds(base, _SCC)], idxd_v)
                copies = [pltpu.make_async_copy(
                    tab_hbm.at[idxs_v.at[pl.ds(j * 128, 128)]],
                    rows_v.at[pl.ds(j * 128, 128)], sem)
                    for j in range(_SCC // 128)]
                for c in copies:
                    c.start()
                for g in range(_SCC // 16):
                    si = idxs_v[pl.ds(g * 16, 16)] * 2
                    di = idxd_v[pl.ds(g * 16, 16)] * 2 + _P2N // 2
                    off = g * 128 + lane8
                    plsc.store_scatter(pbuf_v, [off],
                                       plsc.load_gather(ptab_v, [si]))
                    plsc.store_scatter(pbuf_v, [off + 1],
                                       plsc.load_gather(ptab_v, [si + 1]))
                    plsc.store_scatter(pbuf_v, [off + 2],
                                       plsc.load_gather(ptab_v, [di]))
                    plsc.store_scatter(pbuf_v, [off + 3],
                                       plsc.load_gather(ptab_v, [di + 1]))

                pltpu.sync_copy(pbuf_v, gpos_out.at[pl.ds(base * 8, _SCC * 8)])
                for c in copies:
                    c.wait()
                pltpu.sync_copy(rows_v, gsrc_out.at[pl.ds(base, _SCC)])
            return None

        lax.fori_loop(0, n_steps, step, None)

        if count_dst:
            def cstep(k, _):
                chunk = k * 16 + sid

                @pl.when(chunk < n_chunks)
                def _():
                    pltpu.sync_copy(dst_hbm.at[pl.ds(chunk * _SCC, _SCC)],
                                    idxd_v)
                    for g in range(_SCC // 16):
                        dv = idxd_v[pl.ds(g * 16, 16)]
                        yc = dv - cid * _HRNG
                        yc = jnp.where((yc >= 0) & (yc < _HRNG), yc,
                                       _HRNG + sid)
                        crun, clast = plsc.scan_count(yc)
                        plsc.addupdate_scatter(cnt_v, [yc],
                                               crun.astype(jnp.float32),
                                               mask=clast)
                return None

            lax.fori_loop(0, (n_chunks + 15) // 16, cstep, None)
            pltpu.sync_copy(cnt_v.at[pl.ds(0, _HRNG)],
                            stage.at[pl.ds(sid * _HRNG, _HRNG)])
            plsc.subcore_barrier()

            def fzstep(i, _):
                facc_v[pl.ds(i * 16, 16)] = jnp.zeros((16,), jnp.float32)
                return None

            lax.fori_loop(0, crows // 16, fzstep, None)
            for t in range(16):
                pltpu.sync_copy(
                    stage.at[pl.ds(t * _HRNG + sid * crows, crows)], tmp_v)

                def astep(i, _):
                    facc_v[pl.ds(i * 16, 16)] = (facc_v[pl.ds(i * 16, 16)]
                                                 + tmp_v[pl.ds(i * 16, 16)])
                    return None

                lax.fori_loop(0, crows // 16, astep, None)
            pltpu.sync_copy(
                facc_v, cnt_out.at[pl.ds(cid * _HRNG + sid * crows, crows)])

    outs = [jax.ShapeDtypeStruct((e, 128), jnp.float32),
            jax.ShapeDtypeStruct((e * 8,), jnp.float32)]
    scratch = [
        pltpu.VMEM((_SCC, 128), jnp.float32),
        pltpu.VMEM((_SCC * 8,), jnp.float32),
        pltpu.VMEM((_SCC,), jnp.int32),
        pltpu.VMEM((_SCC,), jnp.int32),
        pltpu.VMEM((_P2N,), jnp.float32),
    ]
    if count_dst:
        outs.append(jax.ShapeDtypeStruct((_NPAD,), jnp.float32))
        scratch += [
            pltpu.VMEM((_HRNG + 16,), jnp.float32),
            pltpu.VMEM((crows,), jnp.float32),
            pltpu.VMEM((crows,), jnp.float32),
            pltpu.VMEM_SHARED((16 * _HRNG,), jnp.float32),
        ]
    scratch.append(pltpu.SemaphoreType.DMA)
    return pl.kernel(
        body,
        out_type=tuple(outs),
        mesh=plsc.VectorSubcoreMesh(core_axis_name="c", subcore_axis_name="s"),
        compiler_params=pltpu.CompilerParams(needs_layout_passes=False),
        scratch_types=scratch,
    )(feat_tab, pos2, src1d, dst1d)


_HRNG = _NPAD // 2
_ACCR = _HRNG + 128


def _scatter_sum2(m_a, dst_a, m_s, dst_s, zeros_hbm):
    e = m_a.shape[0]
    n_chunks = e // _SCB
    n_steps = (n_chunks + 15) // 16
    zrows = _ACCR // 16
    orows = _HRNG // 16
    jrows = _SCB // 128

    def body(ma_hbm, da_hbm, ms_hbm, ds_hbm, z_hbm, out_a, out_s,
             rows0, rows1, idxa0, idxa1, idx2_v, acc, lsem0, lsem1, *_):
        rowsb = (rows0, rows1)
        idxb = (idxa0, idxa1)
        lsem = (lsem0, lsem1)
        cid = lax.axis_index("c")
        sid = lax.axis_index("s")
        lo = cid * _HRNG
        garbage = _HRNG + sid * 8

        def zero_acc():
            pltpu.sync_copy(z_hbm.at[pl.ds(sid * zrows, zrows)],
                            acc.at[pl.ds(sid * zrows, zrows)])

        def run_set(m_hbm, dst_hbm, out):
            zero_acc()
            plsc.subcore_barrier()

            def copies(k, b):
                chunk = k * 16 + sid
                return chunk, [
                    pltpu.make_async_copy(
                        dst_hbm.at[pl.ds(chunk * jrows, jrows)],
                        idxb[b], lsem[b]),
                    pltpu.make_async_copy(
                        m_hbm.at[pl.ds(chunk * _SCB, _SCB)],
                        rowsb[b], lsem[b]),
                ]

            def load(k, b):
                chunk, cs = copies(k, b)

                @pl.when(chunk < n_chunks)
                def _():
                    for c in cs:
                        c.start()

            def proc(k, b):
                chunk, cs = copies(k, b)

                @pl.when(chunk < n_chunks)
                def _():
                    for c in cs:
                        c.wait()
                    for j in range(jrows):
                        for l in range(8):
                            x = idxb[b][j, pl.ds(l * 16, 16)]
                            y = x - lo
                            ok = (y >= 0) & (y < _HRNG)
                            idx2_v[j, pl.ds(l * 16, 16)] = jnp.where(ok, y,
                                                                     garbage)
                        pltpu.sync_copy(rowsb[b].at[pl.ds(j * 128, 128)],
                                        acc.at[idx2_v.at[j]], add=True)

            load(0, 0)

            def outer(k0, _):
                for b in range(2):
                    k = k0 * 2 + b
                    load(k + 1, (b + 1) % 2)
                    proc(k, b)
                return None

            lax.fori_loop(0, (n_steps + 1) // 2, outer, None)
            plsc.subcore_barrier()
            pltpu.sync_copy(acc.at[pl.ds(sid * orows, orows)],
                            out.at[pl.ds(lo + sid * orows, orows)])
            plsc.subcore_barrier()

        run_set(ma_hbm, da_hbm, out_a)
        run_set(ms_hbm, ds_hbm, out_s)

    return pl.kernel(
        body,
        out_type=(jax.ShapeDtypeStruct((_NPAD, 128), jnp.float32),
                  jax.ShapeDtypeStruct((_NPAD, 128), jnp.float32)),
        mesh=plsc.VectorSubcoreMesh(core_axis_name="c", subcore_axis_name="s"),
        scratch_types=[
            pltpu.VMEM((_SCB, 128), jnp.float32),
            pltpu.VMEM((_SCB, 128), jnp.float32),
            pltpu.VMEM((_SCB // 128, 128), jnp.int32),
            pltpu.VMEM((_SCB // 128, 128), jnp.int32),
            pltpu.VMEM((_SCB // 128, 128), jnp.int32),
            pltpu.VMEM_SHARED((_ACCR, 128), jnp.float32),
            pltpu.SemaphoreType.DMA,
            pltpu.SemaphoreType.DMA,
        ],
    )(m_a, dst_a, m_s, dst_s, zeros_hbm)


def _edge_weights(params, name):
    w0 = params[f'{name}_W0']
    w0s = w0[:, 5:133].T
    w0d = jnp.concatenate([w0[:, 0:4],
                           jnp.zeros((256, 4), jnp.float32)], axis=1).T
    w0x = w0[:, 4:5].T
    return (w0s.astype(jnp.bfloat16), w0d.astype(jnp.bfloat16), w0x,
            params[f'{name}_b0'][None, :],
            params[f'{name}_W1'].T.astype(jnp.bfloat16),
            params[f'{name}_b1'][None, :],
            params[f'{name}_W2'].T.astype(jnp.bfloat16),
            params[f'{name}_b2'][None, :])


def kernel(h, u, pos_s, pos_a, a2s_src, a2s_dst, a2s_dis,
           s2s_src, s2s_dst, s2s_dis, params):
    n = h.shape[0]
    e = a2s_src.shape[0]
    zeros_hbm = jnp.zeros((_ACCR, 128), jnp.float32)
    pad2n = jnp.zeros((_P2N // 2 - 2 * n,), jnp.float32)
    pos_sf = pos_s.reshape(-1)
    pos2_a = jnp.concatenate([pos_a.reshape(-1), pad2n, pos_sf, pad2n])
    pos2_s = jnp.concatenate([pos_sf, pad2n, pos_sf, pad2n])

    ga_src, ga_pos = _gather_edges(u, pos2_a, a2s_src, a2s_dst)
    m_a = _edge_mlp(ga_src, ga_pos.reshape(e, 8), a2s_dis,
                    *_edge_weights(params, 'u2h'))
    gs_src, gs_pos, cnt2 = _gather_edges(h, pos2_s, s2s_src, s2s_dst,
                                         count_dst=True)
    m_s = _edge_mlp(gs_src, gs_pos.reshape(e, 8), s2s_dis,
                    *_edge_weights(params, 'h2h'))
    sum_a, sum_s = _scatter_sum2(m_a, a2s_dst.reshape(e // 128, 128),
                                 m_s, s2s_dst.reshape(e // 128, 128), zeros_hbm)
    cnt = cnt2[:n, None]

    w0 = params['upd_W0']
    ph = jnp.concatenate([pos_s, h], axis=1)
    return _node_mlp(ph, sum_a, sum_s, cnt,
                     w0[:, 0:130].T.astype(jnp.bfloat16),
                     w0[:, 130:258].T.astype(jnp.bfloat16),
                     w0[:, 258:386].T.astype(jnp.bfloat16),
                     params['upd_b0'][None, :],
                     params['upd_W1'].T.astype(jnp.bfloat16),
                     params['upd_b1'][None, :],
                     params['upd_W2'].T.astype(jnp.bfloat16),
                     params['upd_b2'][None, :])


# speedup vs baseline: 3.7211x; 1.0253x over previous
import functools

import jax
from jax import lax
import jax.numpy as jnp
from jax.experimental import pallas as pl
from jax.experimental.pallas import tpu as pltpu
from jax.experimental.pallas import tpu_sc as plsc

_BE = 6400
_BN = 2000
_SRCW = 144
_DSTW = 16
_NPAD = 10240
_SCC = 512
_SCB = 256


def _edge_mlp_body(gsrc_ref, gdst_ref, dis_ref, w0s_ref, w0d_ref, w0x_ref,
                   b0_ref, w1_ref, b1_ref, w2_ref, b2_ref, out_ref):
    gs = gsrc_ref[...].astype(jnp.bfloat16)
    gd = gdst_ref[...].astype(jnp.bfloat16)
    pre = jnp.dot(gs, w0s_ref[...], preferred_element_type=jnp.float32)
    pre = pre + jnp.dot(gd, w0d_ref[...], preferred_element_type=jnp.float32)
    pre = pre + dis_ref[...] * w0x_ref[...] + b0_ref[...]
    x = jnp.tanh(pre).astype(jnp.bfloat16)
    x = jnp.dot(x, w1_ref[...], preferred_element_type=jnp.float32) + b1_ref[...]
    x = jnp.tanh(x).astype(jnp.bfloat16)
    out_ref[...] = (jnp.dot(x, w2_ref[...], preferred_element_type=jnp.float32)
                    + b2_ref[...])


def _edge_mlp(gsrc, gdst, dis, w0s, w0d, w0x, b0, w1, b1, w2, b2, be=_BE):
    e = gsrc.shape[0]
    grid = (e // be,)
    wspec = lambda a: pl.BlockSpec(a.shape, lambda i: (0,) * a.ndim)
    return pl.pallas_call(
        _edge_mlp_body,
        grid=grid,
        in_specs=[
            pl.BlockSpec((be, 128), lambda i: (i, 0)),
            pl.BlockSpec((be, 8), lambda i: (i, 0)),
            pl.BlockSpec((be, 1), lambda i: (i, 0)),
            wspec(w0s), wspec(w0d), wspec(w0x), wspec(b0),
            wspec(w1), wspec(b1), wspec(w2), wspec(b2),
        ],
        out_specs=pl.BlockSpec((be, 128), lambda i: (i, 0)),
        out_shape=jax.ShapeDtypeStruct((e, 128), jnp.float32),
    )(gsrc, gdst, dis, w0s, w0d, w0x, b0, w1, b1, w2, b2)


def _node_mlp_body(ph_ref, su_ref, ss_ref, cnt_ref,
                   w0a_ref, w0b_ref, w0c_ref,
                   b0_ref, w1_ref, b1_ref, w2_ref, b2_ref, out_ref):
    ph = ph_ref[...].astype(jnp.bfloat16)
    su = su_ref[...].astype(jnp.bfloat16)
    ss = ss_ref[...]
    cnt = jnp.maximum(cnt_ref[...], 1.0)
    mh = (ss / cnt).astype(jnp.bfloat16)
    pre = jnp.dot(ph, w0a_ref[...], preferred_element_type=jnp.float32)
    pre = pre + jnp.dot(su, w0b_ref[...], preferred_element_type=jnp.float32)
    pre = pre + jnp.dot(mh, w0c_ref[...], preferred_element_type=jnp.float32)
    pre = pre + b0_ref[...]
    x = jnp.tanh(pre).astype(jnp.bfloat16)
    x = jnp.dot(x, w1_ref[...], preferred_element_type=jnp.float32) + b1_ref[...]
    x = jnp.tanh(x).astype(jnp.bfloat16)
    out_ref[...] = (jnp.dot(x, w2_ref[...], preferred_element_type=jnp.float32)
                    + b2_ref[...])


def _node_mlp(ph, su, ss, cnt, w0a, w0b, w0c, b0, w1, b1, w2, b2, bn=_BN):
    n = ph.shape[0]
    grid = (n // bn,)
    wspec = lambda a: pl.BlockSpec(a.shape, lambda i: (0,) * a.ndim)
    part = pl.BlockSpec((bn, 128), lambda i: (i, 0))
    return pl.pallas_call(
        _node_mlp_body,
        grid=grid,
        in_specs=[
            pl.BlockSpec((bn, ph.shape[1]), lambda i: (i, 0)),
            part, part,
            pl.BlockSpec((bn, 1), lambda i: (i, 0)),
            wspec(w0a), wspec(w0b), wspec(w0c), wspec(b0),
            wspec(w1), wspec(b1), wspec(w2), wspec(b2),
        ],
        out_specs=pl.BlockSpec((bn, 128), lambda i: (i, 0)),
        out_shape=jax.ShapeDtypeStruct((n, 128), jnp.float32),
    )(ph, su, ss, cnt, w0a, w0b, w0c, b0, w1, b1, w2, b2)


_P2N = 40960


def _gather_edges(feat_tab, pos2, src1d, dst1d, count_dst=False):
    e = src1d.shape[0]
    n_chunks = e // _SCC
    n_steps = (n_chunks + 31) // 32
    crows = _HRNG // 16

    def body(tab_hbm, pos2_hbm, src_hbm, dst_hbm, gsrc_out, gpos_out,
             *rest):
        if count_dst:
            (cnt_out, rows_v, pbuf_v, idxs_v, idxd_v, ptab_v,
             cnt_v, tmp_v, facc_v, stage, sem) = rest
        else:
            rows_v, pbuf_v, idxs_v, idxd_v, ptab_v, sem = rest
        cid = lax.axis_index("c")
        sid = lax.axis_index("s")
        wid = sid * 2 + cid
        pltpu.sync_copy(pos2_hbm, ptab_v)

        def zstep(i, _):
            pbuf_v[pl.ds(i * 16, 16)] = jnp.zeros((16,), jnp.float32)
            return None

        lax.fori_loop(0, _SCC * 8 // 16, zstep, None)
        if count_dst:
            def czstep(i, _):
                cnt_v[pl.ds(i * 16, 16)] = jnp.zeros((16,), jnp.float32)
                return None

            lax.fori_loop(0, (_HRNG + 16) // 16, czstep, None)
        lane8 = jax.lax.iota(jnp.int32, 16) * 8

        def step(k, _):
            chunk = k * 32 + wid

            @pl.when(chunk < n_chunks)
            def _():
                base = chunk * _SCC
                pltpu.sync_copy(src_hbm.at[pl.ds(base, _SCC)], idxs_v)
                pltpu.sync_copy(dst_hbm.at[pl.ds(base, _SCC)], idxd_v)
                copies = [pltpu.make_async_copy(
                    tab_hbm.at[idxs_v.at[pl.ds(j * 128, 128)]],
                    rows_v.at[pl.ds(j * 128, 128)], sem)
                    for j in range(_SCC // 128)]
                for c in copies:
                    c.start()
                for g in range(_SCC // 16):
                    si = idxs_v[pl.ds(g * 16, 16)] * 2
                    di = idxd_v[pl.ds(g * 16, 16)] * 2 + _P2N // 2
                    off = g * 128 + lane8
                    plsc.store_scatter(pbuf_v, [off],
                                       plsc.load_gather(ptab_v, [si]))
                    plsc.store_scatter(pbuf_v, [off + 1],
                                       plsc.load_gather(ptab_v, [si + 1]))
                    plsc.store_scatter(pbuf_v, [off + 2],
                                       plsc.load_gather(ptab_v, [di]))
                    plsc.store_scatter(pbuf_v, [off + 3],
                                       plsc.load_gather(ptab_v, [di + 1]))

                pltpu.sync_copy(pbuf_v, gpos_out.at[pl.ds(base * 8, _SCC * 8)])
                for c in copies:
                    c.wait()
                pltpu.sync_copy(rows_v, gsrc_out.at[pl.ds(base, _SCC)])
            return None

        lax.fori_loop(0, n_steps, step, None)

        if count_dst:
            def cstep(k, _):
                chunk = k * 16 + sid

                @pl.when(chunk < n_chunks)
                def _():
                    pltpu.sync_copy(dst_hbm.at[pl.ds(chunk * _SCC, _SCC)],
                                    idxd_v)
                    for g in range(_SCC // 16):
                        dv = idxd_v[pl.ds(g * 16, 16)]
                        yc = dv - cid * _HRNG
                        yc = jnp.where((yc >= 0) & (yc < _HRNG), yc,
                                       _HRNG + sid)
                        crun, clast = plsc.scan_count(yc)
                        plsc.addupdate_scatter(cnt_v, [yc],
                                               crun.astype(jnp.float32),
                                               mask=clast)
                return None

            lax.fori_loop(0, (n_chunks + 15) // 16, cstep, None)
            pltpu.sync_copy(cnt_v.at[pl.ds(0, _HRNG)],
                            stage.at[pl.ds(sid * _HRNG, _HRNG)])
            plsc.subcore_barrier()

            def fzstep(i, _):
                facc_v[pl.ds(i * 16, 16)] = jnp.zeros((16,), jnp.float32)
                return None

            lax.fori_loop(0, crows // 16, fzstep, None)
            for t in range(16):
                pltpu.sync_copy(
                    stage.at[pl.ds(t * _HRNG + sid * crows, crows)], tmp_v)

                def astep(i, _):
                    facc_v[pl.ds(i * 16, 16)] = (facc_v[pl.ds(i * 16, 16)]
                                                 + tmp_v[pl.ds(i * 16, 16)])
                    return None

                lax.fori_loop(0, crows // 16, astep, None)
            pltpu.sync_copy(
                facc_v, cnt_out.at[pl.ds(cid * _HRNG + sid * crows, crows)])

    outs = [jax.ShapeDtypeStruct((e, 128), jnp.float32),
            jax.ShapeDtypeStruct((e * 8,), jnp.float32)]
    scratch = [
        pltpu.VMEM((_SCC, 128), jnp.float32),
        pltpu.VMEM((_SCC * 8,), jnp.float32),
        pltpu.VMEM((_SCC,), jnp.int32),
        pltpu.VMEM((_SCC,), jnp.int32),
        pltpu.VMEM((_P2N,), jnp.float32),
    ]
    if count_dst:
        outs.append(jax.ShapeDtypeStruct((_NPAD,), jnp.float32))
        scratch += [
            pltpu.VMEM((_HRNG + 16,), jnp.float32),
            pltpu.VMEM((crows,), jnp.float32),
            pltpu.VMEM((crows,), jnp.float32),
            pltpu.VMEM_SHARED((16 * _HRNG,), jnp.float32),
        ]
    scratch.append(pltpu.SemaphoreType.DMA)
    return pl.kernel(
        body,
        out_type=tuple(outs),
        mesh=plsc.VectorSubcoreMesh(core_axis_name="c", subcore_axis_name="s"),
        compiler_params=pltpu.CompilerParams(needs_layout_passes=False),
        scratch_types=scratch,
    )(feat_tab, pos2, src1d, dst1d)


_HRNG = _NPAD // 2
_ACCR = _HRNG + 128


def _scatter_sum2(m_a, dst_a, m_s, dst_s, zeros_hbm):
    e = m_a.shape[0]
    n_chunks = e // _SCB
    n_steps = (n_chunks + 15) // 16
    zrows = _ACCR // 16
    orows = _HRNG // 16
    jrows = _SCB // 128

    def body(ma_hbm, da_hbm, ms_hbm, ds_hbm, z_hbm, out_a, out_s,
             rows0, rows1, idxa0, idxa1, idx2_v, acc, lsem0, lsem1, *_):
        rowsb = (rows0, rows1)
        idxb = (idxa0, idxa1)
        lsem = (lsem0, lsem1)
        cid = lax.axis_index("c")
        sid = lax.axis_index("s")
        lo = cid * _HRNG
        garbage = _HRNG + sid * 8

        def zero_acc():
            pltpu.sync_copy(z_hbm.at[pl.ds(sid * zrows, zrows)],
                            acc.at[pl.ds(sid * zrows, zrows)])

        def run_set(m_hbm, dst_hbm, out):
            zero_acc()
            plsc.subcore_barrier()

            def copies(k, b):
                chunk = k * 16 + sid
                return chunk, [
                    pltpu.make_async_copy(
                        dst_hbm.at[pl.ds(chunk * jrows, jrows)],
                        idxb[b], lsem[b]),
                    pltpu.make_async_copy(
                        m_hbm.at[pl.ds(chunk * _SCB, _SCB)],
                        rowsb[b], lsem[b]),
                ]

            def load(k, b):
                chunk, cs = copies(k, b)

                @pl.when(chunk < n_chunks)
                def _():
                    for c in cs:
                        c.start()

            def proc(k, b):
                chunk, cs = copies(k, b)

                @pl.when(chunk < n_chunks)
                def _():
                    for c in cs:
                        c.wait()
                    for j in range(jrows):
                        for l in range(8):
                            x = idxb[b][j, pl.ds(l * 16, 16)]
                            y = x - lo
                            ok = (y >= 0) & (y < _HRNG)
                            idx2_v[j, pl.ds(l * 16, 16)] = jnp.where(ok, y,
                                                                     garbage)
                        pltpu.sync_copy(rowsb[b].at[pl.ds(j * 128, 128)],
                                        acc.at[idx2_v.at[j]], add=True)

            load(0, 0)

            def outer(k0, _):
                for b in range(2):
                    k = k0 * 2 + b
                    load(k + 1, (b + 1) % 2)
                    proc(k, b)
                return None

            lax.fori_loop(0, (n_steps + 1) // 2, outer, None)
            plsc.subcore_barrier()
            pltpu.sync_copy(acc.at[pl.ds(sid * orows, orows)],
                            out.at[pl.ds(lo + sid * orows, orows)])
            plsc.subcore_barrier()

        run_set(ma_hbm, da_hbm, out_a)
        run_set(ms_hbm, ds_hbm, out_s)

    return pl.kernel(
        body,
        out_type=(jax.ShapeDtypeStruct((_NPAD, 128), jnp.float32),
                  jax.ShapeDtypeStruct((_NPAD, 128), jnp.float32)),
        mesh=plsc.VectorSubcoreMesh(core_axis_name="c", subcore_axis_name="s"),
        scratch_types=[
            pltpu.VMEM((_SCB, 128), jnp.float32),
            pltpu.VMEM((_SCB, 128), jnp.float32),
            pltpu.VMEM((_SCB // 128, 128), jnp.int32),
            pltpu.VMEM((_SCB // 128, 128), jnp.int32),
            pltpu.VMEM((_SCB // 128, 128), jnp.int32),
            pltpu.VMEM_SHARED((_ACCR, 128), jnp.float32),
            pltpu.SemaphoreType.DMA,
            pltpu.SemaphoreType.DMA,
        ],
    )(m_a, dst_a, m_s, dst_s, zeros_hbm)


def _edge_weights(params, name):
    w0 = params[f'{name}_W0']
    w0s = w0[:, 5:133].T
    w0d = jnp.concatenate([w0[:, 0:4],
                           jnp.zeros((256, 4), jnp.float32)], axis=1).T
    w0x = w0[:, 4:5].T
    return (w0s.astype(jnp.bfloat16), w0d.astype(jnp.bfloat16), w0x,
            params[f'{name}_b0'][None, :],
            params[f'{name}_W1'].T.astype(jnp.bfloat16),
            params[f'{name}_b1'][None, :],
            params[f'{name}_W2'].T.astype(jnp.bfloat16),
            params[f'{name}_b2'][None, :])


def kernel(h, u, pos_s, pos_a, a2s_src, a2s_dst, a2s_dis,
           s2s_src, s2s_dst, s2s_dis, params):
    n = h.shape[0]
    e = a2s_src.shape[0]
    zeros_hbm = jnp.zeros((_ACCR, 128), jnp.float32)
    pad2n = jnp.zeros((_P2N // 2 - 2 * n,), jnp.float32)
    pos_sf = pos_s.reshape(-1)
    pos2_a = jnp.concatenate([pos_a.reshape(-1), pad2n, pos_sf, pad2n])
    pos2_s = jnp.concatenate([pos_sf, pad2n, pos_sf, pad2n])

    ga_src, ga_pos = _gather_edges(u, pos2_a, a2s_src, a2s_dst)
    m_a = _edge_mlp(ga_src, ga_pos.reshape(e, 8), a2s_dis,
                    *_edge_weights(params, 'u2h'))
    gs_src, gs_pos, cnt2 = _gather_edges(h, pos2_s, s2s_src, s2s_dst,
                                         count_dst=True)
    m_s = _edge_mlp(gs_src, gs_pos.reshape(e, 8), s2s_dis,
                    *_edge_weights(params, 'h2h'))
    sum_a, sum_s = _scatter_sum2(m_a, a2s_dst.reshape(e // 128, 128),
                                 m_s, s2s_dst.reshape(e // 128, 128), zeros_hbm)
    cnt = cnt2[:n, None]

    w0 = params['upd_W0']
    ph = jnp.concatenate([pos_s, h], axis=1)
    return _node_mlp(ph, sum_a, sum_s, cnt,
                     w0[:, 0:130].T.astype(jnp.bfloat16),
                     w0[:, 130:258].T.astype(jnp.bfloat16),
                     w0[:, 258:386].T.astype(jnp.bfloat16),
                     params['upd_b0'][None, :],
                     params['upd_W1'].T.astype(jnp.bfloat16),
                     params['upd_b1'][None, :],
                     params['upd_W2'].T.astype(jnp.bfloat16),
                     params['upd_b2'][None, :])


# speedup vs baseline: 3.7224x; 1.0004x over previous
import functools

import jax
from jax import lax
import jax.numpy as jnp
from jax.experimental import pallas as pl
from jax.experimental.pallas import tpu as pltpu
from jax.experimental.pallas import tpu_sc as plsc

_BE = 6400
_BN = 2000
_SRCW = 144
_DSTW = 16
_NPAD = 10240
_SCC = 512
_SCB = 256


def _edge_mlp_body(gsrc_ref, gdst_ref, dis_ref, w0s_ref, w0d_ref, w0x_ref,
                   b0_ref, w1_ref, b1_ref, w2_ref, b2_ref, out_ref):
    gs = gsrc_ref[...].astype(jnp.bfloat16)
    gd = gdst_ref[...].astype(jnp.bfloat16)
    pre = jnp.dot(gs, w0s_ref[...], preferred_element_type=jnp.float32)
    pre = pre + jnp.dot(gd, w0d_ref[...], preferred_element_type=jnp.float32)
    pre = pre + dis_ref[...] * w0x_ref[...] + b0_ref[...]
    x = jnp.tanh(pre).astype(jnp.bfloat16)
    x = jnp.dot(x, w1_ref[...], preferred_element_type=jnp.float32) + b1_ref[...]
    x = jnp.tanh(x).astype(jnp.bfloat16)
    out_ref[...] = (jnp.dot(x, w2_ref[...], preferred_element_type=jnp.float32)
                    + b2_ref[...])


def _edge_mlp(gsrc, gdst, dis, w0s, w0d, w0x, b0, w1, b1, w2, b2, be=_BE):
    e = gsrc.shape[0]
    grid = (e // be,)
    wspec = lambda a: pl.BlockSpec(a.shape, lambda i: (0,) * a.ndim)
    return pl.pallas_call(
        _edge_mlp_body,
        grid=grid,
        in_specs=[
            pl.BlockSpec((be, 128), lambda i: (i, 0)),
            pl.BlockSpec((be, 8), lambda i: (i, 0)),
            pl.BlockSpec((be, 1), lambda i: (i, 0)),
            wspec(w0s), wspec(w0d), wspec(w0x), wspec(b0),
            wspec(w1), wspec(b1), wspec(w2), wspec(b2),
        ],
        out_specs=pl.BlockSpec((be, 128), lambda i: (i, 0)),
        out_shape=jax.ShapeDtypeStruct((e, 128), jnp.float32),
    )(gsrc, gdst, dis, w0s, w0d, w0x, b0, w1, b1, w2, b2)


def _node_mlp_body(ph_ref, su_ref, ss_ref, cnt_ref,
                   w0a_ref, w0b_ref, w0c_ref,
                   b0_ref, w1_ref, b1_ref, w2_ref, b2_ref, out_ref):
    ph = ph_ref[...].astype(jnp.bfloat16)
    su = su_ref[...].astype(jnp.bfloat16)
    ss = ss_ref[...]
    cnt = jnp.maximum(cnt_ref[...], 1.0)
    mh = (ss / cnt).astype(jnp.bfloat16)
    pre = jnp.dot(ph, w0a_ref[...], preferred_element_type=jnp.float32)
    pre = pre + jnp.dot(su, w0b_ref[...], preferred_element_type=jnp.float32)
    pre = pre + jnp.dot(mh, w0c_ref[...], preferred_element_type=jnp.float32)
    pre = pre + b0_ref[...]
    x = jnp.tanh(pre).astype(jnp.bfloat16)
    x = jnp.dot(x, w1_ref[...], preferred_element_type=jnp.float32) + b1_ref[...]
    x = jnp.tanh(x).astype(jnp.bfloat16)
    out_ref[...] = (jnp.dot(x, w2_ref[...], preferred_element_type=jnp.float32)
                    + b2_ref[...])


def _node_mlp(ph, su, ss, cnt, w0a, w0b, w0c, b0, w1, b1, w2, b2, bn=_BN):
    n = ph.shape[0]
    grid = (n // bn,)
    wspec = lambda a: pl.BlockSpec(a.shape, lambda i: (0,) * a.ndim)
    part = pl.BlockSpec((bn, 128), lambda i: (i, 0))
    return pl.pallas_call(
        _node_mlp_body,
        grid=grid,
        in_specs=[
            pl.BlockSpec((bn, ph.shape[1]), lambda i: (i, 0)),
            part, part,
            pl.BlockSpec((bn, 1), lambda i: (i, 0)),
            wspec(w0a), wspec(w0b), wspec(w0c), wspec(b0),
            wspec(w1), wspec(b1), wspec(w2), wspec(b2),
        ],
        out_specs=pl.BlockSpec((bn, 128), lambda i: (i, 0)),
        out_shape=jax.ShapeDtypeStruct((n, 128), jnp.float32),
    )(ph, su, ss, cnt, w0a, w0b, w0c, b0, w1, b1, w2, b2)


_P2N = 40960


def _gather_edges(feat_tab, pos2, src1d, dst1d, count_dst=False):
    e = src1d.shape[0]
    n_chunks = e // _SCC
    n_steps = (n_chunks + 31) // 32
    crows = _HRNG // 16

    def body(tab_hbm, pos2_hbm, src_hbm, dst_hbm, gsrc_out, gpos_out,
             *rest):
        if count_dst:
            (cnt_out, rows_v, pbuf_v, idxs_v, idxd_v, ptab_v,
             cnt_v, tmp_v, facc_v, stage, sem) = rest
        else:
            rows_v, pbuf_v, idxs_v, idxd_v, ptab_v, sem = rest
        cid = lax.axis_index("c")
        sid = lax.axis_index("s")
        wid = sid * 2 + cid
        pltpu.sync_copy(pos2_hbm, ptab_v)

        def zstep(i, _):
            pbuf_v[pl.ds(i * 16, 16)] = jnp.zeros((16,), jnp.float32)
            return None

        lax.fori_loop(0, _SCC * 8 // 16, zstep, None)
        if count_dst:
            def czstep(i, _):
                cnt_v[pl.ds(i * 16, 16)] = jnp.zeros((16,), jnp.float32)
                return None

            lax.fori_loop(0, (_HRNG + 16) // 16, czstep, None)
        lane8 = jax.lax.iota(jnp.int32, 16) * 8

        def step(k, _):
            chunk = k * 32 + wid

            @pl.when(chunk < n_chunks)
            def _():
                base = chunk * _SCC
                pltpu.sync_copy(src_hbm.at[pl.ds(base, _SCC)], idxs_v)
                pltpu.sync_copy(dst_hbm.at[pl.ds(base, _SCC)], idxd_v)
                copies = [pltpu.make_async_copy(
                    tab_hbm.at[idxs_v.at[pl.ds(j * 128, 128)]],
                    rows_v.at[pl.ds(j * 128, 128)], sem)
                    for j in range(_SCC // 128)]
                for c in copies:
                    c.start()
                for g in range(_SCC // 16):
                    si = idxs_v[pl.ds(g * 16, 16)] * 2
                    di = idxd_v[pl.ds(g * 16, 16)] * 2 + _P2N // 2
                    off = g * 128 + lane8
                    plsc.store_scatter(pbuf_v, [off],
                                       plsc.load_gather(ptab_v, [si]))
                    plsc.store_scatter(pbuf_v, [off + 1],
                                       plsc.load_gather(ptab_v, [si + 1]))
                    plsc.store_scatter(pbuf_v, [off + 2],
                                       plsc.load_gather(ptab_v, [di]))
                    plsc.store_scatter(pbuf_v, [off + 3],
                                       plsc.load_gather(ptab_v, [di + 1]))

                pltpu.sync_copy(pbuf_v, gpos_out.at[pl.ds(base * 8, _SCC * 8)])
                for c in copies:
                    c.wait()
                pltpu.sync_copy(rows_v, gsrc_out.at[pl.ds(base, _SCC)])
            return None

        lax.fori_loop(0, n_steps, step, None)

        if count_dst:
            def cstep(k, _):
                chunk = k * 16 + sid

                @pl.when(chunk < n_chunks)
                def _():
                    pltpu.sync_copy(dst_hbm.at[pl.ds(chunk * _SCC, _SCC)],
                                    idxd_v)
                    for g in range(_SCC // 16):
                        dv = idxd_v[pl.ds(g * 16, 16)]
                        yc = dv - cid * _HRNG
                        yc = jnp.where((yc >= 0) & (yc < _HRNG), yc,
                                       _HRNG + sid)
                        crun, clast = plsc.scan_count(yc)
                        plsc.addupdate_scatter(cnt_v, [yc],
                                               crun.astype(jnp.float32),
                                               mask=clast)
                return None

            lax.fori_loop(0, (n_chunks + 15) // 16, cstep, None)
            pltpu.sync_copy(cnt_v.at[pl.ds(0, _HRNG)],
                            stage.at[pl.ds(sid * _HRNG, _HRNG)])
            plsc.subcore_barrier()

            def fzstep(i, _):
                facc_v[pl.ds(i * 16, 16)] = jnp.zeros((16,), jnp.float32)
                return None

            lax.fori_loop(0, crows // 16, fzstep, None)
            for t in range(16):
                pltpu.sync_copy(
                    stage.at[pl.ds(t * _HRNG + sid * crows, crows)], tmp_v)

                def astep(i, _):
                    facc_v[pl.ds(i * 16, 16)] = (facc_v[pl.ds(i * 16, 16)]
                                                 + tmp_v[pl.ds(i * 16, 16)])
                    return None

                lax.fori_loop(0, crows // 16, astep, None)
            pltpu.sync_copy(
                facc_v, cnt_out.at[pl.ds(cid * _HRNG + sid * crows, crows)])

    outs = [jax.ShapeDtypeStruct((e, 128), jnp.float32),
            jax.ShapeDtypeStruct((e * 8,), jnp.float32)]
    scratch = [
        pltpu.VMEM((_SCC, 128), jnp.float32),
        pltpu.VMEM((_SCC * 8,), jnp.float32),
        pltpu.VMEM((_SCC,), jnp.int32),
        pltpu.VMEM((_SCC,), jnp.int32),
        pltpu.VMEM((_P2N,), jnp.float32),
    ]
    if count_dst:
        outs.append(jax.ShapeDtypeStruct((_NPAD,), jnp.float32))
        scratch += [
            pltpu.VMEM((_HRNG + 16,), jnp.float32),
            pltpu.VMEM((crows,), jnp.float32),
            pltpu.VMEM((crows,), jnp.float32),
            pltpu.VMEM_SHARED((16 * _HRNG,), jnp.float32),
        ]
    scratch.append(pltpu.SemaphoreType.DMA)
    return pl.kernel(
        body,
        out_type=tuple(outs),
        mesh=plsc.VectorSubcoreMesh(core_axis_name="c", subcore_axis_name="s"),
        compiler_params=pltpu.CompilerParams(needs_layout_passes=False),
        scratch_types=scratch,
    )(feat_tab, pos2, src1d, dst1d)


_HRNG = _NPAD // 2
_ACCR = _HRNG + 128


def _scatter_sum2(m_a, dst_a, m_s, dst_s, zeros_hbm):
    e = m_a.shape[0]
    n_chunks = e // _SCB
    n_steps = (n_chunks + 15) // 16
    zrows = _ACCR // 16
    orows = _HRNG // 16
    jrows = _SCB // 128

    n_total = 2 * ((n_steps + 1) // 2)

    def body(ma_hbm, da_hbm, ms_hbm, ds_hbm, z_hbm, out_a, out_s,
             rows0, rows1, idxa0, idxa1, idx20, idx21, acc,
             lsem0, lsem1, ssem0, ssem1, *_):
        rowsb = (rows0, rows1)
        idxb = (idxa0, idxa1)
        idx2b = (idx20, idx21)
        lsem = (lsem0, lsem1)
        ssem = (ssem0, ssem1)
        cid = lax.axis_index("c")
        sid = lax.axis_index("s")
        lo = cid * _HRNG
        garbage = _HRNG + sid * 8

        def zero_acc():
            pltpu.sync_copy(z_hbm.at[pl.ds(sid * zrows, zrows)],
                            acc.at[pl.ds(sid * zrows, zrows)])

        def run_set(m_hbm, dst_hbm, out):
            zero_acc()
            plsc.subcore_barrier()

            def copies(k, b):
                chunk = k * 16 + sid
                return chunk, [
                    pltpu.make_async_copy(
                        dst_hbm.at[pl.ds(chunk * jrows, jrows)],
                        idxb[b], lsem[b]),
                    pltpu.make_async_copy(
                        m_hbm.at[pl.ds(chunk * _SCB, _SCB)],
                        rowsb[b], lsem[b]),
                ]

            def scat_copies(b):
                return [pltpu.make_async_copy(
                    rowsb[b].at[pl.ds(j * 128, 128)],
                    acc.at[idx2b[b].at[j]], ssem[b])
                    for j in range(jrows)]

            def drain_scat(k, b):
                @pl.when((k >= 2) & ((k - 2) * 16 + sid < n_chunks))
                def _():
                    for c in scat_copies(b):
                        c.wait()

            def load(k, b):
                chunk, cs = copies(k, b)
                drain_scat(k, b)

                @pl.when(chunk < n_chunks)
                def _():
                    for c in cs:
                        c.start()

            def proc(k, b):
                chunk, cs = copies(k, b)

                @pl.when(chunk < n_chunks)
                def _():
                    for c in cs:
                        c.wait()
                    for j in range(jrows):
                        for l in range(8):
                            x = idxb[b][j, pl.ds(l * 16, 16)]
                            y = x - lo
                            ok = (y >= 0) & (y < _HRNG)
                            idx2b[b][j, pl.ds(l * 16, 16)] = jnp.where(
                                ok, y, garbage)
                    for j in range(jrows):
                        pltpu.async_copy(rowsb[b].at[pl.ds(j * 128, 128)],
                                         acc.at[idx2b[b].at[j]], ssem[b],
                                         add=True)

            load(0, 0)

            def outer(k0, _):
                for b in range(2):
                    k = k0 * 2 + b
                    load(k + 1, (b + 1) % 2)
                    proc(k, b)
                return None

            lax.fori_loop(0, (n_steps + 1) // 2, outer, None)
            drain_scat(n_total + 1, (n_total - 1) % 2)
            plsc.subcore_barrier()
            pltpu.sync_copy(acc.at[pl.ds(sid * orows, orows)],
                            out.at[pl.ds(lo + sid * orows, orows)])
            plsc.subcore_barrier()

        run_set(ma_hbm, da_hbm, out_a)
        run_set(ms_hbm, ds_hbm, out_s)

    return pl.kernel(
        body,
        out_type=(jax.ShapeDtypeStruct((_NPAD, 128), jnp.float32),
                  jax.ShapeDtypeStruct((_NPAD, 128), jnp.float32)),
        mesh=plsc.VectorSubcoreMesh(core_axis_name="c", subcore_axis_name="s"),
        scratch_types=[
            pltpu.VMEM((_SCB, 128), jnp.float32),
            pltpu.VMEM((_SCB, 128), jnp.float32),
            pltpu.VMEM((_SCB // 128, 128), jnp.int32),
            pltpu.VMEM((_SCB // 128, 128), jnp.int32),
            pltpu.VMEM((_SCB // 128, 128), jnp.int32),
            pltpu.VMEM((_SCB // 128, 128), jnp.int32),
            pltpu.VMEM_SHARED((_ACCR, 128), jnp.float32),
            pltpu.SemaphoreType.DMA,
            pltpu.SemaphoreType.DMA,
            pltpu.SemaphoreType.DMA,
            pltpu.SemaphoreType.DMA,
        ],
    )(m_a, dst_a, m_s, dst_s, zeros_hbm)


def _edge_weights(params, name):
    w0 = params[f'{name}_W0']
    w0s = w0[:, 5:133].T
    w0d = jnp.concatenate([w0[:, 0:4],
                           jnp.zeros((256, 4), jnp.float32)], axis=1).T
    w0x = w0[:, 4:5].T
    return (w0s.astype(jnp.bfloat16), w0d.astype(jnp.bfloat16), w0x,
            params[f'{name}_b0'][None, :],
            params[f'{name}_W1'].T.astype(jnp.bfloat16),
            params[f'{name}_b1'][None, :],
            params[f'{name}_W2'].T.astype(jnp.bfloat16),
            params[f'{name}_b2'][None, :])


def kernel(h, u, pos_s, pos_a, a2s_src, a2s_dst, a2s_dis,
           s2s_src, s2s_dst, s2s_dis, params):
    n = h.shape[0]
    e = a2s_src.shape[0]
    zeros_hbm = jnp.zeros((_ACCR, 128), jnp.float32)
    pad2n = jnp.zeros((_P2N // 2 - 2 * n,), jnp.float32)
    pos_sf = pos_s.reshape(-1)
    pos2_a = jnp.concatenate([pos_a.reshape(-1), pad2n, pos_sf, pad2n])
    pos2_s = jnp.concatenate([pos_sf, pad2n, pos_sf, pad2n])

    ga_src, ga_pos = _gather_edges(u, pos2_a, a2s_src, a2s_dst)
    m_a = _edge_mlp(ga_src, ga_pos.reshape(e, 8), a2s_dis,
                    *_edge_weights(params, 'u2h'))
    gs_src, gs_pos, cnt2 = _gather_edges(h, pos2_s, s2s_src, s2s_dst,
                                         count_dst=True)
    m_s = _edge_mlp(gs_src, gs_pos.reshape(e, 8), s2s_dis,
                    *_edge_weights(params, 'h2h'))
    sum_a, sum_s = _scatter_sum2(m_a, a2s_dst.reshape(e // 128, 128),
                                 m_s, s2s_dst.reshape(e // 128, 128), zeros_hbm)
    cnt = cnt2[:n, None]

    w0 = params['upd_W0']
    ph = jnp.concatenate([pos_s, h], axis=1)
    return _node_mlp(ph, sum_a, sum_s, cnt,
                     w0[:, 0:130].T.astype(jnp.bfloat16),
                     w0[:, 130:258].T.astype(jnp.bfloat16),
                     w0[:, 258:386].T.astype(jnp.bfloat16),
                     params['upd_b0'][None, :],
                     params['upd_W1'].T.astype(jnp.bfloat16),
                     params['upd_b1'][None, :],
                     params['upd_W2'].T.astype(jnp.bfloat16),
                     params['upd_b2'][None, :])


# speedup vs baseline: 3.7268x; 1.0012x over previous
import jax
from jax import lax
import jax.numpy as jnp
from jax.experimental import pallas as pl
from jax.experimental.pallas import tpu as pltpu
from jax.experimental.pallas import tpu_sc as plsc

_BE = 6400
_BN = 2000
_NPAD = 10240
_SCC = 512
_SCB = 256


def _edge_mlp_body(gsrc_ref, gdst_ref, dis_ref, w0s_ref, w0d_ref, w0x_ref,
                   b0_ref, w1_ref, b1_ref, w2_ref, b2_ref, out_ref):
    gs = gsrc_ref[...].astype(jnp.bfloat16)
    gd = gdst_ref[...].astype(jnp.bfloat16)
    pre = jnp.dot(gs, w0s_ref[...], preferred_element_type=jnp.float32)
    pre = pre + jnp.dot(gd, w0d_ref[...], preferred_element_type=jnp.float32)
    pre = pre + dis_ref[...] * w0x_ref[...] + b0_ref[...]
    x = jnp.tanh(pre).astype(jnp.bfloat16)
    x = jnp.dot(x, w1_ref[...], preferred_element_type=jnp.float32) + b1_ref[...]
    x = jnp.tanh(x).astype(jnp.bfloat16)
    out_ref[...] = (jnp.dot(x, w2_ref[...], preferred_element_type=jnp.float32)
                    + b2_ref[...])


def _edge_mlp(gsrc, gdst, dis, w0s, w0d, w0x, b0, w1, b1, w2, b2, be=_BE):
    e = gsrc.shape[0]
    grid = (e // be,)
    wspec = lambda a: pl.BlockSpec(a.shape, lambda i: (0,) * a.ndim)
    return pl.pallas_call(
        _edge_mlp_body,
        grid=grid,
        in_specs=[
            pl.BlockSpec((be, 128), lambda i: (i, 0)),
            pl.BlockSpec((be, 8), lambda i: (i, 0)),
            pl.BlockSpec((be, 1), lambda i: (i, 0)),
            wspec(w0s), wspec(w0d), wspec(w0x), wspec(b0),
            wspec(w1), wspec(b1), wspec(w2), wspec(b2),
        ],
        out_specs=pl.BlockSpec((be, 128), lambda i: (i, 0)),
        out_shape=jax.ShapeDtypeStruct((e, 128), jnp.float32),
    )(gsrc, gdst, dis, w0s, w0d, w0x, b0, w1, b1, w2, b2)


def _node_mlp_body(ph_ref, su_ref, ss_ref, cnt_ref,
                   w0a_ref, w0b_ref, w0c_ref,
                   b0_ref, w1_ref, b1_ref, w2_ref, b2_ref, out_ref):
    ph = ph_ref[...].astype(jnp.bfloat16)
    su = su_ref[...].astype(jnp.bfloat16)
    ss = ss_ref[...]
    cnt = jnp.maximum(cnt_ref[...], 1.0)
    mh = (ss / cnt).astype(jnp.bfloat16)
    pre = jnp.dot(ph, w0a_ref[...], preferred_element_type=jnp.float32)
    pre = pre + jnp.dot(su, w0b_ref[...], preferred_element_type=jnp.float32)
    pre = pre + jnp.dot(mh, w0c_ref[...], preferred_element_type=jnp.float32)
    pre = pre + b0_ref[...]
    x = jnp.tanh(pre).astype(jnp.bfloat16)
    x = jnp.dot(x, w1_ref[...], preferred_element_type=jnp.float32) + b1_ref[...]
    x = jnp.tanh(x).astype(jnp.bfloat16)
    out_ref[...] = (jnp.dot(x, w2_ref[...], preferred_element_type=jnp.float32)
                    + b2_ref[...])


def _node_mlp(ph, su, ss, cnt, w0a, w0b, w0c, b0, w1, b1, w2, b2, bn=_BN):
    n = ph.shape[0]
    grid = (n // bn,)
    wspec = lambda a: pl.BlockSpec(a.shape, lambda i: (0,) * a.ndim)
    part = pl.BlockSpec((bn, 128), lambda i: (i, 0))
    return pl.pallas_call(
        _node_mlp_body,
        grid=grid,
        in_specs=[
            pl.BlockSpec((bn, ph.shape[1]), lambda i: (i, 0)),
            part, part,
            pl.BlockSpec((bn, 1), lambda i: (i, 0)),
            wspec(w0a), wspec(w0b), wspec(w0c), wspec(b0),
            wspec(w1), wspec(b1), wspec(w2), wspec(b2),
        ],
        out_specs=pl.BlockSpec((bn, 128), lambda i: (i, 0)),
        out_shape=jax.ShapeDtypeStruct((n, 128), jnp.float32),
    )(ph, su, ss, cnt, w0a, w0b, w0c, b0, w1, b1, w2, b2)


_P2N = 40960


def _gather_edges(feat_tab, pos2, src1d, dst1d, count_dst=False):
    e = src1d.shape[0]
    n_chunks = e // _SCC
    n_steps = (n_chunks + 31) // 32
    crows = _HRNG // 16

    def body(tab_hbm, pos2_hbm, src_hbm, dst_hbm, gsrc_out, gpos_out,
             *rest):
        if count_dst:
            (cnt_out, rows_v, pbuf_v, idxs_v, idxd_v, ptab_v,
             cnt_v, tmp_v, facc_v, stage, sem) = rest
        else:
            rows_v, pbuf_v, idxs_v, idxd_v, ptab_v, sem = rest
        cid = lax.axis_index("c")
        sid = lax.axis_index("s")
        wid = sid * 2 + cid
        pltpu.sync_copy(pos2_hbm, ptab_v)

        def zstep(i, _):
            pbuf_v[pl.ds(i * 16, 16)] = jnp.zeros((16,), jnp.float32)
            return None

        lax.fori_loop(0, _SCC * 8 // 16, zstep, None)
        if count_dst:
            def czstep(i, _):
                cnt_v[pl.ds(i * 16, 16)] = jnp.zeros((16,), jnp.float32)
                return None

            lax.fori_loop(0, (_HRNG + 16) // 16, czstep, None)
        lane8 = jax.lax.iota(jnp.int32, 16) * 8

        def step(k, _):
            chunk = k * 32 + wid

            @pl.when(chunk < n_chunks)
            def _():
                base = chunk * _SCC
                pltpu.sync_copy(src_hbm.at[pl.ds(base, _SCC)], idxs_v)
                pltpu.sync_copy(dst_hbm.at[pl.ds(base, _SCC)], idxd_v)
                copies = [pltpu.make_async_copy(
                    tab_hbm.at[idxs_v.at[pl.ds(j * 128, 128)]],
                    rows_v.at[pl.ds(j * 128, 128)], sem)
                    for j in range(_SCC // 128)]
                for c in copies:
                    c.start()
                for g in range(_SCC // 16):
                    si = idxs_v[pl.ds(g * 16, 16)] * 2
                    di = idxd_v[pl.ds(g * 16, 16)] * 2 + _P2N // 2
                    off = g * 128 + lane8
                    plsc.store_scatter(pbuf_v, [off],
                                       plsc.load_gather(ptab_v, [si]))
                    plsc.store_scatter(pbuf_v, [off + 1],
                                       plsc.load_gather(ptab_v, [si + 1]))
                    plsc.store_scatter(pbuf_v, [off + 2],
                                       plsc.load_gather(ptab_v, [di]))
                    plsc.store_scatter(pbuf_v, [off + 3],
                                       plsc.load_gather(ptab_v, [di + 1]))

                pltpu.sync_copy(pbuf_v, gpos_out.at[pl.ds(base * 8, _SCC * 8)])
                for c in copies:
                    c.wait()
                pltpu.sync_copy(rows_v, gsrc_out.at[pl.ds(base, _SCC)])
            return None

        lax.fori_loop(0, n_steps, step, None)

        if count_dst:
            def cstep(k, _):
                chunk = k * 16 + sid

                @pl.when(chunk < n_chunks)
                def _():
                    pltpu.sync_copy(dst_hbm.at[pl.ds(chunk * _SCC, _SCC)],
                                    idxd_v)
                    for g in range(_SCC // 16):
                        dv = idxd_v[pl.ds(g * 16, 16)]
                        yc = dv - cid * _HRNG
                        yc = jnp.where((yc >= 0) & (yc < _HRNG), yc,
                                       _HRNG + sid)
                        crun, clast = plsc.scan_count(yc)
                        plsc.addupdate_scatter(cnt_v, [yc],
                                               crun.astype(jnp.float32),
                                               mask=clast)
                return None

            lax.fori_loop(0, (n_chunks + 15) // 16, cstep, None)
            pltpu.sync_copy(cnt_v.at[pl.ds(0, _HRNG)],
                            stage.at[pl.ds(sid * _HRNG, _HRNG)])
            plsc.subcore_barrier()

            def fzstep(i, _):
                facc_v[pl.ds(i * 16, 16)] = jnp.zeros((16,), jnp.float32)
                return None

            lax.fori_loop(0, crows // 16, fzstep, None)
            for t in range(16):
                pltpu.sync_copy(
                    stage.at[pl.ds(t * _HRNG + sid * crows, crows)], tmp_v)

                def astep(i, _):
                    facc_v[pl.ds(i * 16, 16)] = (facc_v[pl.ds(i * 16, 16)]
                                                 + tmp_v[pl.ds(i * 16, 16)])
                    return None

                lax.fori_loop(0, crows // 16, astep, None)
            pltpu.sync_copy(
                facc_v, cnt_out.at[pl.ds(cid * _HRNG + sid * crows, crows)])

    outs = [jax.ShapeDtypeStruct((e, 128), jnp.float32),
            jax.ShapeDtypeStruct((e * 8,), jnp.float32)]
    scratch = [
        pltpu.VMEM((_SCC, 128), jnp.float32),
        pltpu.VMEM((_SCC * 8,), jnp.float32),
        pltpu.VMEM((_SCC,), jnp.int32),
        pltpu.VMEM((_SCC,), jnp.int32),
        pltpu.VMEM((_P2N,), jnp.float32),
    ]
    if count_dst:
        outs.append(jax.ShapeDtypeStruct((_NPAD,), jnp.float32))
        scratch += [
            pltpu.VMEM((_HRNG + 16,), jnp.float32),
            pltpu.VMEM((crows,), jnp.float32),
            pltpu.VMEM((crows,), jnp.float32),
            pltpu.VMEM_SHARED((16 * _HRNG,), jnp.float32),
        ]
    scratch.append(pltpu.SemaphoreType.DMA)
    return pl.kernel(
        body,
        out_type=tuple(outs),
        mesh=plsc.VectorSubcoreMesh(core_axis_name="c", subcore_axis_name="s"),
        compiler_params=pltpu.CompilerParams(needs_layout_passes=False),
        scratch_types=scratch,
    )(feat_tab, pos2, src1d, dst1d)


_HRNG = _NPAD // 2
_ACCR = _HRNG + 128


def _scatter_sum2(m_a, dst_a, m_s, dst_s, zeros_hbm):
    e = m_a.shape[0]
    n_chunks = e // _SCB
    n_steps = (n_chunks + 15) // 16
    zrows = _ACCR // 16
    orows = _HRNG // 16
    jrows = _SCB // 128

    n_total = 2 * ((n_steps + 1) // 2)

    def body(ma_hbm, da_hbm, ms_hbm, ds_hbm, z_hbm, out_a, out_s,
             rows0, rows1, idxa0, idxa1, idx20, idx21, acc,
             lsem0, lsem1, ssem0, ssem1, *_):
        rowsb = (rows0, rows1)
        idxb = (idxa0, idxa1)
        idx2b = (idx20, idx21)
        lsem = (lsem0, lsem1)
        ssem = (ssem0, ssem1)
        cid = lax.axis_index("c")
        sid = lax.axis_index("s")
        lo = cid * _HRNG
        garbage = _HRNG + sid * 8

        def zero_acc():
            pltpu.sync_copy(z_hbm.at[pl.ds(sid * zrows, zrows)],
                            acc.at[pl.ds(sid * zrows, zrows)])

        def run_set(m_hbm, dst_hbm, out):
            zero_acc()
            plsc.subcore_barrier()

            def copies(k, b):
                chunk = k * 16 + sid
                return chunk, [
                    pltpu.make_async_copy(
                        dst_hbm.at[pl.ds(chunk * jrows, jrows)],
                        idxb[b], lsem[b]),
                    pltpu.make_async_copy(
                        m_hbm.at[pl.ds(chunk * _SCB, _SCB)],
                        rowsb[b], lsem[b]),
                ]

            def scat_copies(b):
                return [pltpu.make_async_copy(
                    rowsb[b].at[pl.ds(j * 128, 128)],
                    acc.at[idx2b[b].at[j]], ssem[b])
                    for j in range(jrows)]

            def drain_scat(k, b):
                @pl.when((k >= 2) & ((k - 2) * 16 + sid < n_chunks))
                def _():
                    for c in scat_copies(b):
                        c.wait()

            def load(k, b):
                chunk, cs = copies(k, b)
                drain_scat(k, b)

                @pl.when(chunk < n_chunks)
                def _():
                    for c in cs:
                        c.start()

            def proc(k, b):
                chunk, cs = copies(k, b)

                @pl.when(chunk < n_chunks)
                def _():
                    for c in cs:
                        c.wait()
                    for j in range(jrows):
                        for l in range(8):
                            x = idxb[b][j, pl.ds(l * 16, 16)]
                            y = x - lo
                            ok = (y >= 0) & (y < _HRNG)
                            idx2b[b][j, pl.ds(l * 16, 16)] = jnp.where(
                                ok, y, garbage)
                    for j in range(jrows):
                        pltpu.async_copy(rowsb[b].at[pl.ds(j * 128, 128)],
                                         acc.at[idx2b[b].at[j]], ssem[b],
                                         add=True)

            load(0, 0)

            def outer(k0, _):
                for b in range(2):
                    k = k0 * 2 + b
                    load(k + 1, (b + 1) % 2)
                    proc(k, b)
                return None

            lax.fori_loop(0, (n_steps + 1) // 2, outer, None)
            drain_scat(n_total + 1, (n_total - 1) % 2)
            plsc.subcore_barrier()
            pltpu.sync_copy(acc.at[pl.ds(sid * orows, orows)],
                            out.at[pl.ds(lo + sid * orows, orows)])
            plsc.subcore_barrier()

        run_set(ma_hbm, da_hbm, out_a)
        run_set(ms_hbm, ds_hbm, out_s)

    return pl.kernel(
        body,
        out_type=(jax.ShapeDtypeStruct((_NPAD, 128), jnp.float32),
                  jax.ShapeDtypeStruct((_NPAD, 128), jnp.float32)),
        mesh=plsc.VectorSubcoreMesh(core_axis_name="c", subcore_axis_name="s"),
        scratch_types=[
            pltpu.VMEM((_SCB, 128), jnp.float32),
            pltpu.VMEM((_SCB, 128), jnp.float32),
            pltpu.VMEM((_SCB // 128, 128), jnp.int32),
            pltpu.VMEM((_SCB // 128, 128), jnp.int32),
            pltpu.VMEM((_SCB // 128, 128), jnp.int32),
            pltpu.VMEM((_SCB // 128, 128), jnp.int32),
            pltpu.VMEM_SHARED((_ACCR, 128), jnp.float32),
            pltpu.SemaphoreType.DMA,
            pltpu.SemaphoreType.DMA,
            pltpu.SemaphoreType.DMA,
            pltpu.SemaphoreType.DMA,
        ],
    )(m_a, dst_a, m_s, dst_s, zeros_hbm)


def _edge_weights(params, name):
    w0 = params[f'{name}_W0']
    w0s = w0[:, 5:133].T
    w0d = jnp.concatenate([w0[:, 0:4],
                           jnp.zeros((256, 4), jnp.float32)], axis=1).T
    w0x = w0[:, 4:5].T
    return (w0s.astype(jnp.bfloat16), w0d.astype(jnp.bfloat16), w0x,
            params[f'{name}_b0'][None, :],
            params[f'{name}_W1'].T.astype(jnp.bfloat16),
            params[f'{name}_b1'][None, :],
            params[f'{name}_W2'].T.astype(jnp.bfloat16),
            params[f'{name}_b2'][None, :])


def kernel(h, u, pos_s, pos_a, a2s_src, a2s_dst, a2s_dis,
           s2s_src, s2s_dst, s2s_dis, params):
    n = h.shape[0]
    e = a2s_src.shape[0]
    zeros_hbm = jnp.zeros((_ACCR, 128), jnp.float32)
    pad2n = jnp.zeros((_P2N // 2 - 2 * n,), jnp.float32)
    pos_sf = pos_s.reshape(-1)
    pos2_a = jnp.concatenate([pos_a.reshape(-1), pad2n, pos_sf, pad2n])
    pos2_s = jnp.concatenate([pos_sf, pad2n, pos_sf, pad2n])

    ga_src, ga_pos = _gather_edges(u, pos2_a, a2s_src, a2s_dst)
    m_a = _edge_mlp(ga_src, ga_pos.reshape(e, 8), a2s_dis,
                    *_edge_weights(params, 'u2h'))
    gs_src, gs_pos, cnt2 = _gather_edges(h, pos2_s, s2s_src, s2s_dst,
                                         count_dst=True)
    m_s = _edge_mlp(gs_src, gs_pos.reshape(e, 8), s2s_dis,
                    *_edge_weights(params, 'h2h'))
    sum_a, sum_s = _scatter_sum2(m_a, a2s_dst.reshape(e // 128, 128),
                                 m_s, s2s_dst.reshape(e // 128, 128), zeros_hbm)
    cnt = cnt2[:n, None]

    w0 = params['upd_W0']
    ph = jnp.concatenate([pos_s, h], axis=1)
    return _node_mlp(ph, sum_a, sum_s, cnt,
                     w0[:, 0:130].T.astype(jnp.bfloat16),
                     w0[:, 130:258].T.astype(jnp.bfloat16),
                     w0[:, 258:386].T.astype(jnp.bfloat16),
                     params['upd_b0'][None, :],
                     params['upd_W1'].T.astype(jnp.bfloat16),
                     params['upd_b1'][None, :],
                     params['upd_W2'].T.astype(jnp.bfloat16),
                     params['upd_b2'][None, :])
